# Initial kernel scaffold; baseline (speedup 1.0000x reference)
import jax
import jax.numpy as jnp
from jax.experimental import pallas as pl


def kernel(x, edge_index, W1, att_src1, att_dst1, b1, W2, att_src2, att_dst2, b2):
    raise NotImplementedError("write your pallas kernel here")


# speedup vs baseline: 32.4593x; 32.4593x over previous
import jax
import jax.numpy as jnp
from jax import lax
from jax.experimental import pallas as pl
from jax.experimental.pallas import tpu as pltpu
from jax.experimental.pallas import tpu_sc as plsc

F32 = jnp.float32

N_NODES = 50000
E_EDGES = 800000
RB = 400
NC, NS, L = 2, 16, 16
R_HALF = N_NODES // NC
ACC_ROWS = 25088
TRASH = R_HALF
K1 = 64
K2 = 128
EPAD = 800768
EPT = EPAD // NS
ZROWS = ACC_ROWS // NS

_SC_PARAMS = pltpu.CompilerParams(
    use_tc_tiling_on_sc=False, needs_layout_passes=False)


def _cv(v, dtype=jnp.int32):
    return jnp.full((L,), v, dtype)


def _vsel(vals, idx):
    dn = lax.GatherDimensionNumbers(
        offset_dims=(), collapsed_slice_dims=(0,), start_index_map=(0,))
    return lax.gather(vals, idx.reshape(L, 1), dn, slice_sizes=(1,),
                      mode=lax.GatherScatterMode.PROMISE_IN_BOUNDS)


def _tc_a_body(x_ref, w_ref, attsf_ref, attdf_ref, tsrc_ref, tdst_ref):
    xw = jnp.dot(x_ref[...], w_ref[...], preferred_element_type=F32)
    r64 = lax.broadcasted_iota(jnp.int32, (64, 8), 0)
    c8 = lax.broadcasted_iota(jnp.int32, (64, 8), 1)
    S = jnp.where(r64 // 8 == c8, 1.0, 0.0).astype(F32)
    asrc = jnp.dot(xw * attsf_ref[...], S, preferred_element_type=F32)
    adst = jnp.dot(xw * attdf_ref[...], S, preferred_element_type=F32)
    z8 = jnp.zeros((xw.shape[0], 8), F32)
    tsrc_ref[...] = jnp.concatenate([xw, asrc], axis=1)
    tdst_ref[...] = jnp.concatenate([adst, z8], axis=1)


def _tc_a(x, W1, attsf, attdf):
    n, f = x.shape
    return pl.pallas_call(
        _tc_a_body,
        grid=(n // RB,),
        in_specs=[pl.BlockSpec((RB, f), lambda i: (i, 0)),
                  pl.BlockSpec((f, 64), lambda i: (0, 0)),
                  pl.BlockSpec((1, 64), lambda i: (0, 0)),
                  pl.BlockSpec((1, 64), lambda i: (0, 0))],
        out_specs=[pl.BlockSpec((RB, 72), lambda i: (i, 0)),
                   pl.BlockSpec((RB, 16), lambda i: (i, 0))],
        out_shape=[jax.ShapeDtypeStruct((n, 72), F32),
                   jax.ShapeDtypeStruct((n, 16), F32)],
    )(x, W1, attsf, attdf)


def _sc_edge1_body(src_h, dst_h, tsrc_h, tdst_h, onum_h, oden_h,
                   accn_sh, accd_sh, src_v, dst_v, dloc_v, rows_v, adst_v,
                   cnum_v, cden_v, sem1, sem2):
    c = lax.axis_index("c")
    s = lax.axis_index("s")
    lo = c * R_HALF
    lane = lax.iota(jnp.int32, L)
    zv = jnp.zeros((L,), F32)
    lov = jnp.full((L,), lo, jnp.int32)
    zero_i = _cv(0)
    rhalf_v = _cv(R_HALF)
    trash_v = _cv(TRASH)
    eedge_v = _cv(E_EDGES)
    zero_f = _cv(0.0, F32)
    slope_v = _cv(0.2, F32)
    eight_v = _cv(8)
    hrep_idx = [lane // eight_v + _cv(2 * j) for j in range(4)]
    den_mask = lane < eight_v

    def _zrow(r, carry):
        for j in range(4):
            cnum_v[r, pl.ds(j * L, L)] = zv
        plsc.store_scatter(cden_v, [jnp.full((L,), r, jnp.int32), lane], zv,
                           mask=den_mask)
        return carry
    lax.fori_loop(0, K1, _zrow, 0)
    zbase = s * ZROWS
    off = 0
    while off < ZROWS:
        sz = min(K1, ZROWS - off)
        pltpu.sync_copy(cnum_v.at[pl.ds(0, sz)],
                        accn_sh.at[pl.ds(zbase + off, sz)])
        pltpu.sync_copy(cden_v.at[pl.ds(0, sz)],
                        accd_sh.at[pl.ds(zbase + off, sz)])
        off += sz
    plsc.subcore_barrier()

    ebase = s * EPT

    def _chunk(g, carry):
        eo = ebase + g * K1
        pltpu.sync_copy(src_h.at[pl.ds(eo, K1)], src_v)
        pltpu.sync_copy(dst_h.at[pl.ds(eo, K1)], dst_v)
        for i in range(K1 // L):
            d = dst_v[pl.ds(i * L, L)]
            eid = jnp.full((L,), eo + i * L, jnp.int32) + lane
            dl = d - lov
            inr = (dl >= zero_i) & (dl < rhalf_v) & (eid < eedge_v)
            dloc_v[pl.ds(i * L, L)] = jnp.where(inr, dl, trash_v)
        cp1 = pltpu.async_copy(tsrc_h.at[src_v], rows_v, sem1)
        cp2 = pltpu.async_copy(tdst_h.at[dst_v], adst_v, sem2)
        cp1.wait()
        cp2.wait()

        def _edge(e, carry2):
            asrc = rows_v[e, pl.ds(56, L)]
            ad = adst_v[e, pl.ds(0, L)]
            al = _vsel(asrc, lane % eight_v + eight_v) + ad
            al = jnp.where(al >= zero_f, al, al * slope_v)
            w = jnp.exp(al)
            plsc.store_scatter(cden_v, [jnp.full((L,), e, jnp.int32), lane],
                               w, mask=den_mask)
            for j in range(4):
                wr = _vsel(w, hrep_idx[j])
                cnum_v[e, pl.ds(j * L, L)] = rows_v[e, pl.ds(j * L, L)] * wr
            return carry2
        lax.fori_loop(0, K1, _edge, 0)
        pltpu.sync_copy(cnum_v, accn_sh.at[dloc_v], add=True)
        pltpu.sync_copy(cden_v, accd_sh.at[dloc_v], add=True)
        return carry
    lax.fori_loop(0, EPT // K1, _chunk, 0)
    plsc.subcore_barrier()

    @pl.when(s < 5)
    def _copy_out():
        rows = R_HALF // 5
        pltpu.sync_copy(accn_sh.at[pl.ds(s * rows, rows)],
                        onum_h.at[pl.ds(c * R_HALF + s * rows, rows)])
        pltpu.sync_copy(accd_sh.at[pl.ds(s * rows, rows)],
                        oden_h.at[pl.ds(c * R_HALF + s * rows, rows)])


def _sc_edge1(srcp, dstp, tsrc, tdst):
    return pl.kernel(
        _sc_edge1_body,
        out_type=[jax.ShapeDtypeStruct((N_NODES, 64), F32),
                  jax.ShapeDtypeStruct((N_NODES, 8), F32)],
        compiler_params=_SC_PARAMS,
        mesh=plsc.VectorSubcoreMesh(core_axis_name="c", subcore_axis_name="s"),
        scratch_types=[
            pltpu.VMEM_SHARED((ACC_ROWS, 64), F32),
            pltpu.VMEM_SHARED((ACC_ROWS, 8), F32),
            pltpu.VMEM((K1,), jnp.int32),
            pltpu.VMEM((K1,), jnp.int32),
            pltpu.VMEM((K1,), jnp.int32),
            pltpu.VMEM((K1, 72), F32),
            pltpu.VMEM((K1, 16), F32),
            pltpu.VMEM((K1, 64), F32),
            pltpu.VMEM((K1, 8), F32),
            pltpu.SemaphoreType.DMA,
            pltpu.SemaphoreType.DMA,
        ],
    )(srcp, dstp, tsrc, tdst)


def _tc_c_body(tsrc_ref, tdst_ref, an_ref, ad_ref, b1_ref, w2_ref, asf_ref,
               adf_ref, t2_ref):
    ts = tsrc_ref[...]
    xw = ts[:, 0:64]
    aw = ts[:, 64:72] + tdst_ref[...][:, 0:8]
    wself = jnp.exp(jnp.where(aw >= 0.0, aw, aw * 0.2))
    inv = 1.0 / (ad_ref[...] + wself + 1e-16)
    r8 = lax.broadcasted_iota(jnp.int32, (8, 64), 0)
    c64 = lax.broadcasted_iota(jnp.int32, (8, 64), 1)
    R8 = jnp.where(r8 == c64 // 8, 1.0, 0.0).astype(F32)
    wrep = jnp.dot(wself, R8, preferred_element_type=F32)
    invrep = jnp.dot(inv, R8, preferred_element_type=F32)
    h1 = (an_ref[...] + wrep * xw) * invrep + b1_ref[...]
    h1 = jnp.where(h1 > 0.0, h1, jnp.exp(jnp.minimum(h1, 0.0)) - 1.0)
    xw2 = jnp.dot(h1, w2_ref[...], preferred_element_type=F32)
    asrc2 = jnp.sum(xw2 * asf_ref[...], axis=1, keepdims=True)
    adst2 = jnp.sum(xw2 * adf_ref[...], axis=1, keepdims=True)
    z7 = jnp.zeros((xw2.shape[0], 7), F32)
    t2_ref[...] = jnp.concatenate([xw2, asrc2, adst2, z7], axis=1)


def _tc_c(tsrc, tdst, acc1n, acc1d, b1r, W2, asf2, adf2):
    n = tsrc.shape[0]
    return pl.pallas_call(
        _tc_c_body,
        grid=(n // RB,),
        in_specs=[pl.BlockSpec((RB, 72), lambda i: (i, 0)),
                  pl.BlockSpec((RB, 16), lambda i: (i, 0)),
                  pl.BlockSpec((RB, 64), lambda i: (i, 0)),
                  pl.BlockSpec((RB, 8), lambda i: (i, 0)),
                  pl.BlockSpec((1, 64), lambda i: (0, 0)),
                  pl.BlockSpec((64, 7), lambda i: (0, 0)),
                  pl.BlockSpec((1, 7), lambda i: (0, 0)),
                  pl.BlockSpec((1, 7), lambda i: (0, 0))],
        out_specs=pl.BlockSpec((RB, 16), lambda i: (i, 0)),
        out_shape=jax.ShapeDtypeStruct((n, 16), F32),
    )(tsrc, tdst, acc1n, acc1d, b1r, W2, asf2, adf2)


def _sc_edge2_body(src_h, dst_h, t2_h, out_h,
                   acc_sh, src_v, dst_v, dloc_v, rows_v, adst_v, con_v,
                   sem1, sem2):
    c = lax.axis_index("c")
    s = lax.axis_index("s")
    lo = c * R_HALF
    lane = lax.iota(jnp.int32, L)
    zv = jnp.zeros((L,), F32)
    lov = jnp.full((L,), lo, jnp.int32)
    zero_i = _cv(0)
    rhalf_v = _cv(R_HALF)
    trash_v = _cv(TRASH)
    eedge_v = _cv(E_EDGES)
    zero_f = _cv(0.0, F32)
    slope_v = _cv(0.2, F32)
    seven_v = _cv(7)
    eight_v = _cv(8)

    def _zrow(r, carry):
        con_v[r, pl.ds(0, L)] = zv
        return carry
    lax.fori_loop(0, K2, _zrow, 0)
    zbase = s * ZROWS
    off = 0
    while off < ZROWS:
        sz = min(K2, ZROWS - off)
        pltpu.sync_copy(con_v.at[pl.ds(0, sz)],
                        acc_sh.at[pl.ds(zbase + off, sz)])
        off += sz
    plsc.subcore_barrier()

    ebase = s * EPT

    def _chunk(g, carry):
        eo = ebase + g * K2
        pltpu.sync_copy(src_h.at[pl.ds(eo, K2)], src_v)
        pltpu.sync_copy(dst_h.at[pl.ds(eo, K2)], dst_v)
        for i in range(K2 // L):
            d = dst_v[pl.ds(i * L, L)]
            eid = jnp.full((L,), eo + i * L, jnp.int32) + lane
            dl = d - lov
            inr = (dl >= zero_i) & (dl < rhalf_v) & (eid < eedge_v)
            dloc_v[pl.ds(i * L, L)] = jnp.where(inr, dl, trash_v)
        cp1 = pltpu.async_copy(t2_h.at[src_v], rows_v, sem1)
        cp2 = pltpu.async_copy(t2_h.at[dst_v], adst_v, sem2)
        cp1.wait()
        cp2.wait()

        def _edge(e, carry2):
            r = rows_v[e, pl.ds(0, L)]
            rd = adst_v[e, pl.ds(0, L)]
            al = _vsel(r, seven_v) + _vsel(rd, eight_v)
            al = jnp.where(al >= zero_f, al, al * slope_v)
            w = jnp.exp(al)
            con = jnp.where(lane < seven_v, r * w,
                            jnp.where(lane == seven_v, w, zero_f))
            con_v[e, pl.ds(0, L)] = con
            return carry2
        lax.fori_loop(0, K2, _edge, 0)
        pltpu.sync_copy(con_v, acc_sh.at[dloc_v], add=True)
        return carry
    lax.fori_loop(0, EPT // K2, _chunk, 0)
    plsc.subcore_barrier()

    @pl.when(s < 5)
    def _copy_out():
        rows = R_HALF // 5
        pltpu.sync_copy(acc_sh.at[pl.ds(s * rows, rows)],
                        out_h.at[pl.ds(c * R_HALF + s * rows, rows)])


def _sc_edge2(srcp, dstp, t2):
    return pl.kernel(
        _sc_edge2_body,
        out_type=jax.ShapeDtypeStruct((N_NODES, 16), F32),
        compiler_params=_SC_PARAMS,
        mesh=plsc.VectorSubcoreMesh(core_axis_name="c", subcore_axis_name="s"),
        scratch_types=[
            pltpu.VMEM_SHARED((ACC_ROWS, 16), F32),
            pltpu.VMEM((K2,), jnp.int32),
            pltpu.VMEM((K2,), jnp.int32),
            pltpu.VMEM((K2,), jnp.int32),
            pltpu.VMEM((K2, 16), F32),
            pltpu.VMEM((K2, 16), F32),
            pltpu.VMEM((K2, 16), F32),
            pltpu.SemaphoreType.DMA,
            pltpu.SemaphoreType.DMA,
        ],
    )(srcp, dstp, t2)


def _tc_e_body(acc_ref, t2_ref, b2_ref, out_ref):
    ac = acc_ref[...]
    t2 = t2_ref[...]
    a2 = t2[:, 7:8] + t2[:, 8:9]
    w = jnp.exp(jnp.where(a2 >= 0.0, a2, a2 * 0.2))
    num = ac[:, 0:7] + w * t2[:, 0:7]
    den = ac[:, 7:8] + w + 1e-16
    o = num / den + b2_ref[...]
    m = jnp.max(o, axis=1, keepdims=True)
    sh = o - m
    lse = jnp.log(jnp.sum(jnp.exp(sh), axis=1, keepdims=True))
    res = sh - lse
    out_ref[...] = jnp.concatenate(
        [res, jnp.zeros((res.shape[0], 1), F32)], axis=1)


def _tc_e(acc2, t2, b2r):
    n = acc2.shape[0]
    return pl.pallas_call(
        _tc_e_body,
        grid=(n // RB,),
        in_specs=[pl.BlockSpec((RB, 16), lambda i: (i, 0)),
                  pl.BlockSpec((RB, 16), lambda i: (i, 0)),
                  pl.BlockSpec((1, 7), lambda i: (0, 0))],
        out_specs=pl.BlockSpec((RB, 8), lambda i: (i, 0)),
        out_shape=jax.ShapeDtypeStruct((n, 8), F32),
    )(acc2, t2, b2r)


def kernel(x, edge_index, W1, att_src1, att_dst1, b1, W2, att_src2, att_dst2,
           b2):
    attsf = att_src1.reshape(1, 64)
    attdf = att_dst1.reshape(1, 64)
    asf2 = att_src2.reshape(1, 7)
    adf2 = att_dst2.reshape(1, 7)
    b1r = b1.reshape(1, 64)
    b2r = b2.reshape(1, 7)

    pad = EPAD - E_EDGES
    srcp = jnp.concatenate([edge_index[0], jnp.zeros((pad,), jnp.int32)])
    dstp = jnp.concatenate([edge_index[1], jnp.zeros((pad,), jnp.int32)])

    tsrc, tdst = _tc_a(x, W1, attsf, attdf)
    acc1n, acc1d = _sc_edge1(srcp, dstp, tsrc, tdst)
    t2 = _tc_c(tsrc, tdst, acc1n, acc1d, b1r, W2, asf2, adf2)
    acc2 = _sc_edge2(srcp, dstp, t2)
    outp = _tc_e(acc2, t2, b2r)
    return outp[:, :7]


# speedup vs baseline: 48.9713x; 1.5087x over previous
import jax
import jax.numpy as jnp
from jax import lax
from jax.experimental import pallas as pl
from jax.experimental.pallas import tpu as pltpu
from jax.experimental.pallas import tpu_sc as plsc

F32 = jnp.float32

N_NODES = 50000
E_EDGES = 800000
RB = 400
NC, NS, L = 2, 16, 16
R_HALF = N_NODES // NC
ACC_ROWS = 25088
TRASH = R_HALF
K1 = 64
EPT = 50048
NCH = EPT // K1
EPAD = NS * EPT + 2 * K1
ZROWS = ACC_ROWS // NS

_SC_PARAMS = pltpu.CompilerParams(
    use_tc_tiling_on_sc=False, needs_layout_passes=False)


def _cv(v, dtype=jnp.int32):
    return jnp.full((L,), v, dtype)


def _vsel(vals, idx):
    dn = lax.GatherDimensionNumbers(
        offset_dims=(), collapsed_slice_dims=(0,), start_index_map=(0,))
    return lax.gather(vals, idx.reshape(L, 1), dn, slice_sizes=(1,),
                      mode=lax.GatherScatterMode.PROMISE_IN_BOUNDS)


def _tc_a_body(x_ref, w_ref, attsf_ref, attdf_ref, tsrc_ref, tdst_ref):
    xw = jnp.dot(x_ref[...], w_ref[...], preferred_element_type=F32)
    r64 = lax.broadcasted_iota(jnp.int32, (64, 8), 0)
    c8 = lax.broadcasted_iota(jnp.int32, (64, 8), 1)
    S = jnp.where(r64 // 8 == c8, 1.0, 0.0).astype(F32)
    asrc = jnp.dot(xw * attsf_ref[...], S, preferred_element_type=F32)
    adst = jnp.dot(xw * attdf_ref[...], S, preferred_element_type=F32)
    z8 = jnp.zeros((xw.shape[0], 8), F32)
    tsrc_ref[...] = jnp.concatenate([xw, asrc], axis=1)
    tdst_ref[...] = jnp.concatenate([adst, z8], axis=1)


def _tc_a(x, W1, attsf, attdf):
    n, f = x.shape
    return pl.pallas_call(
        _tc_a_body,
        grid=(n // RB,),
        in_specs=[pl.BlockSpec((RB, f), lambda i: (i, 0)),
                  pl.BlockSpec((f, 64), lambda i: (0, 0)),
                  pl.BlockSpec((1, 64), lambda i: (0, 0)),
                  pl.BlockSpec((1, 64), lambda i: (0, 0))],
        out_specs=[pl.BlockSpec((RB, 72), lambda i: (i, 0)),
                   pl.BlockSpec((RB, 16), lambda i: (i, 0))],
        out_shape=[jax.ShapeDtypeStruct((n, 72), F32),
                   jax.ShapeDtypeStruct((n, 16), F32)],
    )(x, W1, attsf, attdf)


def _edge_pipeline(ebase, issue_idx, drain_idx, compute_dloc, issue_gather,
                   drain_gather, compute_chunk, issue_scatter, drain_scatter):
    issue_idx(0, ebase)
    drain_idx(0)
    compute_dloc(0, ebase)
    issue_gather(0)
    issue_idx(1, ebase + K1)

    def _pair(k, carry):
        eo0 = ebase + (2 * k) * K1
        drain_gather(0)

        @pl.when(k > 0)
        def _():
            drain_scatter(1)
        drain_idx(1)
        compute_dloc(1, eo0 + K1)
        issue_gather(1)
        issue_idx(0, eo0 + 2 * K1)
        compute_chunk(0)
        issue_scatter(0)
        drain_gather(1)
        drain_scatter(0)
        drain_idx(0)
        compute_dloc(0, eo0 + 2 * K1)
        issue_gather(0)
        issue_idx(1, eo0 + 3 * K1)
        compute_chunk(1)
        issue_scatter(1)
        return carry
    lax.fori_loop(0, NCH // 2, _pair, 0)
    drain_gather(0)
    drain_idx(1)
    drain_scatter(1)


def _sc_edge1_body(src_h, dst_h, tsrc_h, tdst_h, onum_h, oden_h,
                   accn_sh, accd_sh,
                   src0_v, src1_v, dst0_v, dst1_v, dloc0_v, dloc1_v,
                   rows0_v, rows1_v, adst0_v, adst1_v, cnum_v, cden_v,
                   si0, si1, sg0, sg1, ss):
    c = lax.axis_index("c")
    s = lax.axis_index("s")
    lo = c * R_HALF
    lane = lax.iota(jnp.int32, L)
    zv = jnp.zeros((L,), F32)
    lov = jnp.full((L,), lo, jnp.int32)
    zero_i = _cv(0)
    rhalf_v = _cv(R_HALF)
    trash_v = _cv(TRASH)
    eedge_v = _cv(E_EDGES)
    zero_f = _cv(0.0, F32)
    slope_v = _cv(0.2, F32)
    eight_v = _cv(8)
    hrep_idx = [lane // eight_v + _cv(2 * j) for j in range(4)]
    den_mask = lane < eight_v
    srcs = (src0_v, src1_v)
    dsts = (dst0_v, dst1_v)
    dlocs = (dloc0_v, dloc1_v)
    rows = (rows0_v, rows1_v)
    adsts = (adst0_v, adst1_v)
    sis = (si0, si1)
    sgs = (sg0, sg1)

    def _zrow(r, carry):
        for j in range(4):
            cnum_v[r, pl.ds(j * L, L)] = zv
        plsc.store_scatter(cden_v, [jnp.full((L,), r, jnp.int32), lane], zv,
                           mask=den_mask)
        return carry
    lax.fori_loop(0, K1, _zrow, 0)
    zbase = s * ZROWS
    off = 0
    while off < ZROWS:
        sz = min(K1, ZROWS - off)
        pltpu.sync_copy(cnum_v.at[pl.ds(0, sz)],
                        accn_sh.at[pl.ds(zbase + off, sz)])
        pltpu.sync_copy(cden_v.at[pl.ds(0, sz)],
                        accd_sh.at[pl.ds(zbase + off, sz)])
        off += sz
    plsc.subcore_barrier()

    def issue_idx(p, off):
        pltpu.async_copy(src_h.at[pl.ds(off, K1)], srcs[p], sis[p])
        pltpu.async_copy(dst_h.at[pl.ds(off, K1)], dsts[p], sis[p])

    def drain_idx(p):
        pltpu.make_async_copy(src_h.at[pl.ds(0, K1)], srcs[p], sis[p]).wait()
        pltpu.make_async_copy(dst_h.at[pl.ds(0, K1)], dsts[p], sis[p]).wait()

    def compute_dloc(p, eo):
        for i in range(K1 // L):
            d = dsts[p][pl.ds(i * L, L)]
            eid = jnp.full((L,), eo + i * L, jnp.int32) + lane
            dl = d - lov
            inr = (dl >= zero_i) & (dl < rhalf_v) & (eid < eedge_v)
            dlocs[p][pl.ds(i * L, L)] = jnp.where(inr, dl, trash_v)

    def issue_gather(p):
        pltpu.async_copy(tsrc_h.at[srcs[p]], rows[p], sgs[p])
        pltpu.async_copy(tdst_h.at[dsts[p]], adsts[p], sgs[p])

    def drain_gather(p):
        pltpu.make_async_copy(tsrc_h.at[srcs[p]], rows[p], sgs[p]).wait()
        pltpu.make_async_copy(tdst_h.at[dsts[p]], adsts[p], sgs[p]).wait()

    def compute_chunk(p):
        rows_v = rows[p]
        adst_v = adsts[p]

        def _edge(e, carry2):
            asrc = rows_v[e, pl.ds(56, L)]
            ad = adst_v[e, pl.ds(0, L)]
            al = _vsel(asrc, lane % eight_v + eight_v) + ad
            al = jnp.where(al >= zero_f, al, al * slope_v)
            w = jnp.exp(al)
            plsc.store_scatter(cden_v, [jnp.full((L,), e, jnp.int32), lane],
                               w, mask=den_mask)
            for j in range(4):
                wr = _vsel(w, hrep_idx[j])
                cnum_v[e, pl.ds(j * L, L)] = rows_v[e, pl.ds(j * L, L)] * wr
            return carry2
        lax.fori_loop(0, K1, _edge, 0)

    def issue_scatter(p):
        pltpu.async_copy(cnum_v, accn_sh.at[dlocs[p]], ss, add=True)
        pltpu.async_copy(cden_v, accd_sh.at[dlocs[p]], ss, add=True)

    def drain_scatter(p):
        pltpu.make_async_copy(cnum_v, accn_sh.at[dlocs[p]], ss).wait()
        pltpu.make_async_copy(cden_v, accd_sh.at[dlocs[p]], ss).wait()

    _edge_pipeline(s * EPT, issue_idx, drain_idx, compute_dloc, issue_gather,
                   drain_gather, compute_chunk, issue_scatter, drain_scatter)
    plsc.subcore_barrier()

    @pl.when(s < 5)
    def _copy_out():
        nrows = R_HALF // 5
        pltpu.sync_copy(accn_sh.at[pl.ds(s * nrows, nrows)],
                        onum_h.at[pl.ds(c * R_HALF + s * nrows, nrows)])
        pltpu.sync_copy(accd_sh.at[pl.ds(s * nrows, nrows)],
                        oden_h.at[pl.ds(c * R_HALF + s * nrows, nrows)])


def _sc_edge1(srcp, dstp, tsrc, tdst):
    return pl.kernel(
        _sc_edge1_body,
        out_type=[jax.ShapeDtypeStruct((N_NODES, 64), F32),
                  jax.ShapeDtypeStruct((N_NODES, 8), F32)],
        compiler_params=_SC_PARAMS,
        mesh=plsc.VectorSubcoreMesh(core_axis_name="c", subcore_axis_name="s"),
        scratch_types=[
            pltpu.VMEM_SHARED((ACC_ROWS, 64), F32),
            pltpu.VMEM_SHARED((ACC_ROWS, 8), F32),
            pltpu.VMEM((K1,), jnp.int32),
            pltpu.VMEM((K1,), jnp.int32),
            pltpu.VMEM((K1,), jnp.int32),
            pltpu.VMEM((K1,), jnp.int32),
            pltpu.VMEM((K1,), jnp.int32),
            pltpu.VMEM((K1,), jnp.int32),
            pltpu.VMEM((K1, 72), F32),
            pltpu.VMEM((K1, 72), F32),
            pltpu.VMEM((K1, 16), F32),
            pltpu.VMEM((K1, 16), F32),
            pltpu.VMEM((K1, 64), F32),
            pltpu.VMEM((K1, 8), F32),
            pltpu.SemaphoreType.DMA,
            pltpu.SemaphoreType.DMA,
            pltpu.SemaphoreType.DMA,
            pltpu.SemaphoreType.DMA,
            pltpu.SemaphoreType.DMA,
        ],
    )(srcp, dstp, tsrc, tdst)


def _tc_c_body(tsrc_ref, tdst_ref, an_ref, ad_ref, b1_ref, w2_ref, asf_ref,
               adf_ref, t2_ref):
    ts = tsrc_ref[...]
    xw = ts[:, 0:64]
    aw = ts[:, 64:72] + tdst_ref[...][:, 0:8]
    wself = jnp.exp(jnp.where(aw >= 0.0, aw, aw * 0.2))
    inv = 1.0 / (ad_ref[...] + wself + 1e-16)
    r8 = lax.broadcasted_iota(jnp.int32, (8, 64), 0)
    c64 = lax.broadcasted_iota(jnp.int32, (8, 64), 1)
    R8 = jnp.where(r8 == c64 // 8, 1.0, 0.0).astype(F32)
    wrep = jnp.dot(wself, R8, preferred_element_type=F32)
    invrep = jnp.dot(inv, R8, preferred_element_type=F32)
    h1 = (an_ref[...] + wrep * xw) * invrep + b1_ref[...]
    h1 = jnp.where(h1 > 0.0, h1, jnp.exp(jnp.minimum(h1, 0.0)) - 1.0)
    xw2 = jnp.dot(h1, w2_ref[...], preferred_element_type=F32)
    asrc2 = jnp.sum(xw2 * asf_ref[...], axis=1, keepdims=True)
    adst2 = jnp.sum(xw2 * adf_ref[...], axis=1, keepdims=True)
    z7 = jnp.zeros((xw2.shape[0], 7), F32)
    t2_ref[...] = jnp.concatenate([xw2, asrc2, adst2, z7], axis=1)


def _tc_c(tsrc, tdst, acc1n, acc1d, b1r, W2, asf2, adf2):
    n = tsrc.shape[0]
    return pl.pallas_call(
        _tc_c_body,
        grid=(n // RB,),
        in_specs=[pl.BlockSpec((RB, 72), lambda i: (i, 0)),
                  pl.BlockSpec((RB, 16), lambda i: (i, 0)),
                  pl.BlockSpec((RB, 64), lambda i: (i, 0)),
                  pl.BlockSpec((RB, 8), lambda i: (i, 0)),
                  pl.BlockSpec((1, 64), lambda i: (0, 0)),
                  pl.BlockSpec((64, 7), lambda i: (0, 0)),
                  pl.BlockSpec((1, 7), lambda i: (0, 0)),
                  pl.BlockSpec((1, 7), lambda i: (0, 0))],
        out_specs=pl.BlockSpec((RB, 16), lambda i: (i, 0)),
        out_shape=jax.ShapeDtypeStruct((n, 16), F32),
    )(tsrc, tdst, acc1n, acc1d, b1r, W2, asf2, adf2)


def _sc_edge2_body(src_h, dst_h, t2_h, out_h,
                   acc_sh,
                   src0_v, src1_v, dst0_v, dst1_v, dloc0_v, dloc1_v,
                   rows0_v, rows1_v, adst0_v, adst1_v, con_v,
                   si0, si1, sg0, sg1, ss):
    c = lax.axis_index("c")
    s = lax.axis_index("s")
    lo = c * R_HALF
    lane = lax.iota(jnp.int32, L)
    zv = jnp.zeros((L,), F32)
    lov = jnp.full((L,), lo, jnp.int32)
    zero_i = _cv(0)
    rhalf_v = _cv(R_HALF)
    trash_v = _cv(TRASH)
    eedge_v = _cv(E_EDGES)
    zero_f = _cv(0.0, F32)
    slope_v = _cv(0.2, F32)
    seven_v = _cv(7)
    eight_v = _cv(8)
    srcs = (src0_v, src1_v)
    dsts = (dst0_v, dst1_v)
    dlocs = (dloc0_v, dloc1_v)
    rows = (rows0_v, rows1_v)
    adsts = (adst0_v, adst1_v)
    sis = (si0, si1)
    sgs = (sg0, sg1)

    def _zrow(r, carry):
        con_v[r, pl.ds(0, L)] = zv
        return carry
    lax.fori_loop(0, K1, _zrow, 0)
    zbase = s * ZROWS
    off = 0
    while off < ZROWS:
        sz = min(K1, ZROWS - off)
        pltpu.sync_copy(con_v.at[pl.ds(0, sz)],
                        acc_sh.at[pl.ds(zbase + off, sz)])
        off += sz
    plsc.subcore_barrier()

    def issue_idx(p, off):
        pltpu.async_copy(src_h.at[pl.ds(off, K1)], srcs[p], sis[p])
        pltpu.async_copy(dst_h.at[pl.ds(off, K1)], dsts[p], sis[p])

    def drain_idx(p):
        pltpu.make_async_copy(src_h.at[pl.ds(0, K1)], srcs[p], sis[p]).wait()
        pltpu.make_async_copy(dst_h.at[pl.ds(0, K1)], dsts[p], sis[p]).wait()

    def compute_dloc(p, eo):
        for i in range(K1 // L):
            d = dsts[p][pl.ds(i * L, L)]
            eid = jnp.full((L,), eo + i * L, jnp.int32) + lane
            dl = d - lov
            inr = (dl >= zero_i) & (dl < rhalf_v) & (eid < eedge_v)
            dlocs[p][pl.ds(i * L, L)] = jnp.where(inr, dl, trash_v)

    def issue_gather(p):
        pltpu.async_copy(t2_h.at[srcs[p]], rows[p], sgs[p])
        pltpu.async_copy(t2_h.at[dsts[p]], adsts[p], sgs[p])

    def drain_gather(p):
        pltpu.make_async_copy(t2_h.at[srcs[p]], rows[p], sgs[p]).wait()
        pltpu.make_async_copy(t2_h.at[dsts[p]], adsts[p], sgs[p]).wait()

    def compute_chunk(p):
        rows_v = rows[p]
        adst_v = adsts[p]

        def _edge(e, carry2):
            r = rows_v[e, pl.ds(0, L)]
            rd = adst_v[e, pl.ds(0, L)]
            al = _vsel(r, seven_v) + _vsel(rd, eight_v)
            al = jnp.where(al >= zero_f, al, al * slope_v)
            w = jnp.exp(al)
            con = jnp.where(lane < seven_v, r * w,
                            jnp.where(lane == seven_v, w, zero_f))
            con_v[e, pl.ds(0, L)] = con
            return carry2
        lax.fori_loop(0, K1, _edge, 0)

    def issue_scatter(p):
        pltpu.async_copy(con_v, acc_sh.at[dlocs[p]], ss, add=True)

    def drain_scatter(p):
        pltpu.make_async_copy(con_v, acc_sh.at[dlocs[p]], ss).wait()

    _edge_pipeline(s * EPT, issue_idx, drain_idx, compute_dloc, issue_gather,
                   drain_gather, compute_chunk, issue_scatter, drain_scatter)
    plsc.subcore_barrier()

    @pl.when(s < 5)
    def _copy_out():
        nrows = R_HALF // 5
        pltpu.sync_copy(acc_sh.at[pl.ds(s * nrows, nrows)],
                        out_h.at[pl.ds(c * R_HALF + s * nrows, nrows)])


def _sc_edge2(srcp, dstp, t2):
    return pl.kernel(
        _sc_edge2_body,
        out_type=jax.ShapeDtypeStruct((N_NODES, 16), F32),
        compiler_params=_SC_PARAMS,
        mesh=plsc.VectorSubcoreMesh(core_axis_name="c", subcore_axis_name="s"),
        scratch_types=[
            pltpu.VMEM_SHARED((ACC_ROWS, 16), F32),
            pltpu.VMEM((K1,), jnp.int32),
            pltpu.VMEM((K1,), jnp.int32),
            pltpu.VMEM((K1,), jnp.int32),
            pltpu.VMEM((K1,), jnp.int32),
            pltpu.VMEM((K1,), jnp.int32),
            pltpu.VMEM((K1,), jnp.int32),
            pltpu.VMEM((K1, 16), F32),
            pltpu.VMEM((K1, 16), F32),
            pltpu.VMEM((K1, 16), F32),
            pltpu.VMEM((K1, 16), F32),
            pltpu.VMEM((K1, 16), F32),
            pltpu.SemaphoreType.DMA,
            pltpu.SemaphoreType.DMA,
            pltpu.SemaphoreType.DMA,
            pltpu.SemaphoreType.DMA,
            pltpu.SemaphoreType.DMA,
        ],
    )(srcp, dstp, t2)


def _tc_e_body(acc_ref, t2_ref, b2_ref, out_ref):
    ac = acc_ref[...]
    t2 = t2_ref[...]
    a2 = t2[:, 7:8] + t2[:, 8:9]
    w = jnp.exp(jnp.where(a2 >= 0.0, a2, a2 * 0.2))
    num = ac[:, 0:7] + w * t2[:, 0:7]
    den = ac[:, 7:8] + w + 1e-16
    o = num / den + b2_ref[...]
    m = jnp.max(o, axis=1, keepdims=True)
    sh = o - m
    lse = jnp.log(jnp.sum(jnp.exp(sh), axis=1, keepdims=True))
    res = sh - lse
    out_ref[...] = jnp.concatenate(
        [res, jnp.zeros((res.shape[0], 1), F32)], axis=1)


def _tc_e(acc2, t2, b2r):
    n = acc2.shape[0]
    return pl.pallas_call(
        _tc_e_body,
        grid=(n // RB,),
        in_specs=[pl.BlockSpec((RB, 16), lambda i: (i, 0)),
                  pl.BlockSpec((RB, 16), lambda i: (i, 0)),
                  pl.BlockSpec((1, 7), lambda i: (0, 0))],
        out_specs=pl.BlockSpec((RB, 8), lambda i: (i, 0)),
        out_shape=jax.ShapeDtypeStruct((n, 8), F32),
    )(acc2, t2, b2r)


def kernel(x, edge_index, W1, att_src1, att_dst1, b1, W2, att_src2, att_dst2,
           b2):
    attsf = att_src1.reshape(1, 64)
    attdf = att_dst1.reshape(1, 64)
    asf2 = att_src2.reshape(1, 7)
    adf2 = att_dst2.reshape(1, 7)
    b1r = b1.reshape(1, 64)
    b2r = b2.reshape(1, 7)

    pad = EPAD - E_EDGES
    srcp = jnp.concatenate([edge_index[0], jnp.zeros((pad,), jnp.int32)])
    dstp = jnp.concatenate([edge_index[1], jnp.zeros((pad,), jnp.int32)])

    tsrc, tdst = _tc_a(x, W1, attsf, attdf)
    acc1n, acc1d = _sc_edge1(srcp, dstp, tsrc, tdst)
    t2 = _tc_c(tsrc, tdst, acc1n, acc1d, b1r, W2, asf2, adf2)
    acc2 = _sc_edge2(srcp, dstp, t2)
    outp = _tc_e(acc2, t2, b2r)
    return outp[:, :7]


# speedup vs baseline: 57.7368x; 1.1790x over previous
import jax
import jax.numpy as jnp
from jax import lax
from jax.experimental import pallas as pl
from jax.experimental.pallas import tpu as pltpu
from jax.experimental.pallas import tpu_sc as plsc

F32 = jnp.float32

N_NODES = 50000
E_EDGES = 800000
RB = 400
NC, NS, L = 2, 16, 16
R_HALF = N_NODES // NC
ACC_ROWS = 25088
TRASH = R_HALF
K1 = 64
EPT = 50048
NCH = EPT // K1
EPAD = NS * EPT + 2 * K1
ZROWS = ACC_ROWS // NS

_SC_PARAMS = pltpu.CompilerParams(
    use_tc_tiling_on_sc=False, needs_layout_passes=False)


def _cv(v, dtype=jnp.int32):
    return jnp.full((L,), v, dtype)


def _vsel(vals, idx):
    dn = lax.GatherDimensionNumbers(
        offset_dims=(), collapsed_slice_dims=(0,), start_index_map=(0,))
    return lax.gather(vals, idx.reshape(L, 1), dn, slice_sizes=(1,),
                      mode=lax.GatherScatterMode.PROMISE_IN_BOUNDS)


def _tc_a_body(x_ref, w_ref, attsf_ref, attdf_ref, tsrc_ref, tdst_ref):
    xw = jnp.dot(x_ref[...], w_ref[...], preferred_element_type=F32)
    r64 = lax.broadcasted_iota(jnp.int32, (64, 8), 0)
    c8 = lax.broadcasted_iota(jnp.int32, (64, 8), 1)
    S = jnp.where(r64 // 8 == c8, 1.0, 0.0).astype(F32)
    asrc = jnp.dot(xw * attsf_ref[...], S, preferred_element_type=F32)
    adst = jnp.dot(xw * attdf_ref[...], S, preferred_element_type=F32)
    z8 = jnp.zeros((xw.shape[0], 8), F32)
    tsrc_ref[...] = jnp.concatenate([xw, asrc], axis=1)
    tdst_ref[...] = jnp.concatenate([adst, z8], axis=1)


def _tc_a(x, W1, attsf, attdf):
    n, f = x.shape
    return pl.pallas_call(
        _tc_a_body,
        grid=(n // RB,),
        in_specs=[pl.BlockSpec((RB, f), lambda i: (i, 0)),
                  pl.BlockSpec((f, 64), lambda i: (0, 0)),
                  pl.BlockSpec((1, 64), lambda i: (0, 0)),
                  pl.BlockSpec((1, 64), lambda i: (0, 0))],
        out_specs=[pl.BlockSpec((RB, 72), lambda i: (i, 0)),
                   pl.BlockSpec((RB, 16), lambda i: (i, 0))],
        out_shape=[jax.ShapeDtypeStruct((n, 72), F32),
                   jax.ShapeDtypeStruct((n, 16), F32)],
    )(x, W1, attsf, attdf)


def _edge_pipeline(ebase, issue_idx, drain_idx, compute_dloc, issue_gather,
                   drain_gather, compute_chunk, issue_scatter, drain_scatter):
    issue_idx(0, ebase)
    drain_idx(0)
    compute_dloc(0, ebase)
    issue_gather(0)
    issue_idx(1, ebase + K1)

    def _pair(k, carry):
        eo0 = ebase + (2 * k) * K1
        drain_gather(0)

        @pl.when(k > 0)
        def _():
            drain_scatter(1)
        drain_idx(1)
        compute_dloc(1, eo0 + K1)
        issue_gather(1)
        issue_idx(0, eo0 + 2 * K1)
        compute_chunk(0)
        issue_scatter(0)
        drain_gather(1)
        drain_scatter(0)
        drain_idx(0)
        compute_dloc(0, eo0 + 2 * K1)
        issue_gather(0)
        issue_idx(1, eo0 + 3 * K1)
        compute_chunk(1)
        issue_scatter(1)
        return carry
    lax.fori_loop(0, NCH // 2, _pair, 0)
    drain_gather(0)
    drain_idx(1)
    drain_scatter(1)


def _sc_edge1_body(src_h, dst_h, tsrc_h, tdst_h, onum_h, oden_h,
                   accn_sh, accd_sh,
                   src0_v, src1_v, dst0_v, dst1_v, dloc0_v, dloc1_v,
                   rows0_v, rows1_v, adst0_v, adst1_v, cnum_v, cden_v,
                   si0, si1, sg0, sg1, ss):
    c = lax.axis_index("c")
    s = lax.axis_index("s")
    lo = c * R_HALF
    lane = lax.iota(jnp.int32, L)
    zv = jnp.zeros((L,), F32)
    lov = jnp.full((L,), lo, jnp.int32)
    zero_i = _cv(0)
    rhalf_v = _cv(R_HALF)
    trash_v = _cv(TRASH)
    eedge_v = _cv(E_EDGES)
    zero_f = _cv(0.0, F32)
    slope_v = _cv(0.2, F32)
    eight_v = _cv(8)
    hrep_idx = [lane // eight_v + _cv(2 * j) for j in range(4)]
    den_mask = lane < eight_v
    srcs = (src0_v, src1_v)
    dsts = (dst0_v, dst1_v)
    dlocs = (dloc0_v, dloc1_v)
    rows = (rows0_v, rows1_v)
    adsts = (adst0_v, adst1_v)
    sis = (si0, si1)
    sgs = (sg0, sg1)

    def _zrow(r, carry):
        for j in range(4):
            cnum_v[r, pl.ds(j * L, L)] = zv
        plsc.store_scatter(cden_v, [jnp.full((L,), r, jnp.int32), lane], zv,
                           mask=den_mask)
        return carry
    lax.fori_loop(0, K1, _zrow, 0)
    zbase = s * ZROWS
    off = 0
    while off < ZROWS:
        sz = min(K1, ZROWS - off)
        pltpu.sync_copy(cnum_v.at[pl.ds(0, sz)],
                        accn_sh.at[pl.ds(zbase + off, sz)])
        pltpu.sync_copy(cden_v.at[pl.ds(0, sz)],
                        accd_sh.at[pl.ds(zbase + off, sz)])
        off += sz
    plsc.subcore_barrier()

    def issue_idx(p, off):
        pltpu.async_copy(src_h.at[pl.ds(off, K1)], srcs[p], sis[p])
        pltpu.async_copy(dst_h.at[pl.ds(off, K1)], dsts[p], sis[p])

    def drain_idx(p):
        pltpu.make_async_copy(src_h.at[pl.ds(0, K1)], srcs[p], sis[p]).wait()
        pltpu.make_async_copy(dst_h.at[pl.ds(0, K1)], dsts[p], sis[p]).wait()

    def compute_dloc(p, eo):
        for i in range(K1 // L):
            d = dsts[p][pl.ds(i * L, L)]
            eid = jnp.full((L,), eo + i * L, jnp.int32) + lane
            dl = d - lov
            inr = (dl >= zero_i) & (dl < rhalf_v) & (eid < eedge_v)
            dlocs[p][pl.ds(i * L, L)] = jnp.where(inr, dl, trash_v)

    def issue_gather(p):
        pltpu.async_copy(tsrc_h.at[srcs[p]], rows[p], sgs[p])
        pltpu.async_copy(tdst_h.at[dsts[p]], adsts[p], sgs[p])

    def drain_gather(p):
        pltpu.make_async_copy(tsrc_h.at[srcs[p]], rows[p], sgs[p]).wait()
        pltpu.make_async_copy(tdst_h.at[dsts[p]], adsts[p], sgs[p]).wait()

    def compute_chunk(p):
        rows_v = rows[p]
        adst_v = adsts[p]
        perm8 = lane % eight_v + eight_v

        def _edge4(q, carry2):
            es = [4 * q + u for u in range(4)]
            ws = []
            for e in es:
                asrc = rows_v[e, pl.ds(56, L)]
                ad = adst_v[e, pl.ds(0, L)]
                al = _vsel(asrc, perm8) + ad
                al = jnp.where(al >= zero_f, al, al * slope_v)
                ws.append(jnp.exp(al))
            for e, w in zip(es, ws):
                plsc.store_scatter(cden_v,
                                   [jnp.full((L,), e, jnp.int32), lane],
                                   w, mask=den_mask)
            for j in range(4):
                for e, w in zip(es, ws):
                    wr = _vsel(w, hrep_idx[j])
                    cnum_v[e, pl.ds(j * L, L)] = rows_v[e, pl.ds(j * L, L)] * wr
            return carry2
        lax.fori_loop(0, K1 // 4, _edge4, 0)

    def issue_scatter(p):
        pltpu.async_copy(cnum_v, accn_sh.at[dlocs[p]], ss, add=True)
        pltpu.async_copy(cden_v, accd_sh.at[dlocs[p]], ss, add=True)

    def drain_scatter(p):
        pltpu.make_async_copy(cnum_v, accn_sh.at[dlocs[p]], ss).wait()
        pltpu.make_async_copy(cden_v, accd_sh.at[dlocs[p]], ss).wait()

    _edge_pipeline(s * EPT, issue_idx, drain_idx, compute_dloc, issue_gather,
                   drain_gather, compute_chunk, issue_scatter, drain_scatter)
    plsc.subcore_barrier()

    @pl.when(s < 5)
    def _copy_out():
        nrows = R_HALF // 5
        pltpu.sync_copy(accn_sh.at[pl.ds(s * nrows, nrows)],
                        onum_h.at[pl.ds(c * R_HALF + s * nrows, nrows)])
        pltpu.sync_copy(accd_sh.at[pl.ds(s * nrows, nrows)],
                        oden_h.at[pl.ds(c * R_HALF + s * nrows, nrows)])


def _sc_edge1(srcp, dstp, tsrc, tdst):
    return pl.kernel(
        _sc_edge1_body,
        out_type=[jax.ShapeDtypeStruct((N_NODES, 64), F32),
                  jax.ShapeDtypeStruct((N_NODES, 8), F32)],
        compiler_params=_SC_PARAMS,
        mesh=plsc.VectorSubcoreMesh(core_axis_name="c", subcore_axis_name="s"),
        scratch_types=[
            pltpu.VMEM_SHARED((ACC_ROWS, 64), F32),
            pltpu.VMEM_SHARED((ACC_ROWS, 8), F32),
            pltpu.VMEM((K1,), jnp.int32),
            pltpu.VMEM((K1,), jnp.int32),
            pltpu.VMEM((K1,), jnp.int32),
            pltpu.VMEM((K1,), jnp.int32),
            pltpu.VMEM((K1,), jnp.int32),
            pltpu.VMEM((K1,), jnp.int32),
            pltpu.VMEM((K1, 72), F32),
            pltpu.VMEM((K1, 72), F32),
            pltpu.VMEM((K1, 16), F32),
            pltpu.VMEM((K1, 16), F32),
            pltpu.VMEM((K1, 64), F32),
            pltpu.VMEM((K1, 8), F32),
            pltpu.SemaphoreType.DMA,
            pltpu.SemaphoreType.DMA,
            pltpu.SemaphoreType.DMA,
            pltpu.SemaphoreType.DMA,
            pltpu.SemaphoreType.DMA,
        ],
    )(srcp, dstp, tsrc, tdst)


def _tc_c_body(tsrc_ref, tdst_ref, an_ref, ad_ref, b1_ref, w2_ref, asf_ref,
               adf_ref, t2_ref):
    ts = tsrc_ref[...]
    xw = ts[:, 0:64]
    aw = ts[:, 64:72] + tdst_ref[...][:, 0:8]
    wself = jnp.exp(jnp.where(aw >= 0.0, aw, aw * 0.2))
    inv = 1.0 / (ad_ref[...] + wself + 1e-16)
    r8 = lax.broadcasted_iota(jnp.int32, (8, 64), 0)
    c64 = lax.broadcasted_iota(jnp.int32, (8, 64), 1)
    R8 = jnp.where(r8 == c64 // 8, 1.0, 0.0).astype(F32)
    wrep = jnp.dot(wself, R8, preferred_element_type=F32)
    invrep = jnp.dot(inv, R8, preferred_element_type=F32)
    h1 = (an_ref[...] + wrep * xw) * invrep + b1_ref[...]
    h1 = jnp.where(h1 > 0.0, h1, jnp.exp(jnp.minimum(h1, 0.0)) - 1.0)
    xw2 = jnp.dot(h1, w2_ref[...], preferred_element_type=F32)
    asrc2 = jnp.sum(xw2 * asf_ref[...], axis=1, keepdims=True)
    adst2 = jnp.sum(xw2 * adf_ref[...], axis=1, keepdims=True)
    z7 = jnp.zeros((xw2.shape[0], 7), F32)
    t2_ref[...] = jnp.concatenate([xw2, asrc2, adst2, z7], axis=1)


def _tc_c(tsrc, tdst, acc1n, acc1d, b1r, W2, asf2, adf2):
    n = tsrc.shape[0]
    return pl.pallas_call(
        _tc_c_body,
        grid=(n // RB,),
        in_specs=[pl.BlockSpec((RB, 72), lambda i: (i, 0)),
                  pl.BlockSpec((RB, 16), lambda i: (i, 0)),
                  pl.BlockSpec((RB, 64), lambda i: (i, 0)),
                  pl.BlockSpec((RB, 8), lambda i: (i, 0)),
                  pl.BlockSpec((1, 64), lambda i: (0, 0)),
                  pl.BlockSpec((64, 7), lambda i: (0, 0)),
                  pl.BlockSpec((1, 7), lambda i: (0, 0)),
                  pl.BlockSpec((1, 7), lambda i: (0, 0))],
        out_specs=pl.BlockSpec((RB, 16), lambda i: (i, 0)),
        out_shape=jax.ShapeDtypeStruct((n, 16), F32),
    )(tsrc, tdst, acc1n, acc1d, b1r, W2, asf2, adf2)


def _sc_edge2_body(src_h, dst_h, t2_h, out_h,
                   acc_sh,
                   src0_v, src1_v, dst0_v, dst1_v, dloc0_v, dloc1_v,
                   rows0_v, rows1_v, adst0_v, adst1_v, con_v,
                   si0, si1, sg0, sg1, ss):
    c = lax.axis_index("c")
    s = lax.axis_index("s")
    lo = c * R_HALF
    lane = lax.iota(jnp.int32, L)
    zv = jnp.zeros((L,), F32)
    lov = jnp.full((L,), lo, jnp.int32)
    zero_i = _cv(0)
    rhalf_v = _cv(R_HALF)
    trash_v = _cv(TRASH)
    eedge_v = _cv(E_EDGES)
    zero_f = _cv(0.0, F32)
    slope_v = _cv(0.2, F32)
    seven_v = _cv(7)
    eight_v = _cv(8)
    srcs = (src0_v, src1_v)
    dsts = (dst0_v, dst1_v)
    dlocs = (dloc0_v, dloc1_v)
    rows = (rows0_v, rows1_v)
    adsts = (adst0_v, adst1_v)
    sis = (si0, si1)
    sgs = (sg0, sg1)

    def _zrow(r, carry):
        con_v[r, pl.ds(0, L)] = zv
        return carry
    lax.fori_loop(0, K1, _zrow, 0)
    zbase = s * ZROWS
    off = 0
    while off < ZROWS:
        sz = min(K1, ZROWS - off)
        pltpu.sync_copy(con_v.at[pl.ds(0, sz)],
                        acc_sh.at[pl.ds(zbase + off, sz)])
        off += sz
    plsc.subcore_barrier()

    def issue_idx(p, off):
        pltpu.async_copy(src_h.at[pl.ds(off, K1)], srcs[p], sis[p])
        pltpu.async_copy(dst_h.at[pl.ds(off, K1)], dsts[p], sis[p])

    def drain_idx(p):
        pltpu.make_async_copy(src_h.at[pl.ds(0, K1)], srcs[p], sis[p]).wait()
        pltpu.make_async_copy(dst_h.at[pl.ds(0, K1)], dsts[p], sis[p]).wait()

    def compute_dloc(p, eo):
        for i in range(K1 // L):
            d = dsts[p][pl.ds(i * L, L)]
            eid = jnp.full((L,), eo + i * L, jnp.int32) + lane
            dl = d - lov
            inr = (dl >= zero_i) & (dl < rhalf_v) & (eid < eedge_v)
            dlocs[p][pl.ds(i * L, L)] = jnp.where(inr, dl, trash_v)

    def issue_gather(p):
        pltpu.async_copy(t2_h.at[srcs[p]], rows[p], sgs[p])
        pltpu.async_copy(t2_h.at[dsts[p]], adsts[p], sgs[p])

    def drain_gather(p):
        pltpu.make_async_copy(t2_h.at[srcs[p]], rows[p], sgs[p]).wait()
        pltpu.make_async_copy(t2_h.at[dsts[p]], adsts[p], sgs[p]).wait()

    def compute_chunk(p):
        rows_v = rows[p]
        adst_v = adsts[p]

        def _edge4(q, carry2):
            for u in range(4):
                e = 4 * q + u
                r = rows_v[e, pl.ds(0, L)]
                rd = adst_v[e, pl.ds(0, L)]
                al = _vsel(r, seven_v) + _vsel(rd, eight_v)
                al = jnp.where(al >= zero_f, al, al * slope_v)
                w = jnp.exp(al)
                con = jnp.where(lane < seven_v, r * w,
                                jnp.where(lane == seven_v, w, zero_f))
                con_v[e, pl.ds(0, L)] = con
            return carry2
        lax.fori_loop(0, K1 // 4, _edge4, 0)

    def issue_scatter(p):
        pltpu.async_copy(con_v, acc_sh.at[dlocs[p]], ss, add=True)

    def drain_scatter(p):
        pltpu.make_async_copy(con_v, acc_sh.at[dlocs[p]], ss).wait()

    _edge_pipeline(s * EPT, issue_idx, drain_idx, compute_dloc, issue_gather,
                   drain_gather, compute_chunk, issue_scatter, drain_scatter)
    plsc.subcore_barrier()

    @pl.when(s < 5)
    def _copy_out():
        nrows = R_HALF // 5
        pltpu.sync_copy(acc_sh.at[pl.ds(s * nrows, nrows)],
                        out_h.at[pl.ds(c * R_HALF + s * nrows, nrows)])


def _sc_edge2(srcp, dstp, t2):
    return pl.kernel(
        _sc_edge2_body,
        out_type=jax.ShapeDtypeStruct((N_NODES, 16), F32),
        compiler_params=_SC_PARAMS,
        mesh=plsc.VectorSubcoreMesh(core_axis_name="c", subcore_axis_name="s"),
        scratch_types=[
            pltpu.VMEM_SHARED((ACC_ROWS, 16), F32),
            pltpu.VMEM((K1,), jnp.int32),
            pltpu.VMEM((K1,), jnp.int32),
            pltpu.VMEM((K1,), jnp.int32),
            pltpu.VMEM((K1,), jnp.int32),
            pltpu.VMEM((K1,), jnp.int32),
            pltpu.VMEM((K1,), jnp.int32),
            pltpu.VMEM((K1, 16), F32),
            pltpu.VMEM((K1, 16), F32),
            pltpu.VMEM((K1, 16), F32),
            pltpu.VMEM((K1, 16), F32),
            pltpu.VMEM((K1, 16), F32),
            pltpu.SemaphoreType.DMA,
            pltpu.SemaphoreType.DMA,
            pltpu.SemaphoreType.DMA,
            pltpu.SemaphoreType.DMA,
            pltpu.SemaphoreType.DMA,
        ],
    )(srcp, dstp, t2)


def _tc_e_body(acc_ref, t2_ref, b2_ref, out_ref):
    ac = acc_ref[...]
    t2 = t2_ref[...]
    a2 = t2[:, 7:8] + t2[:, 8:9]
    w = jnp.exp(jnp.where(a2 >= 0.0, a2, a2 * 0.2))
    num = ac[:, 0:7] + w * t2[:, 0:7]
    den = ac[:, 7:8] + w + 1e-16
    o = num / den + b2_ref[...]
    m = jnp.max(o, axis=1, keepdims=True)
    sh = o - m
    lse = jnp.log(jnp.sum(jnp.exp(sh), axis=1, keepdims=True))
    res = sh - lse
    out_ref[...] = jnp.concatenate(
        [res, jnp.zeros((res.shape[0], 1), F32)], axis=1)


def _tc_e(acc2, t2, b2r):
    n = acc2.shape[0]
    return pl.pallas_call(
        _tc_e_body,
        grid=(n // RB,),
        in_specs=[pl.BlockSpec((RB, 16), lambda i: (i, 0)),
                  pl.BlockSpec((RB, 16), lambda i: (i, 0)),
                  pl.BlockSpec((1, 7), lambda i: (0, 0))],
        out_specs=pl.BlockSpec((RB, 8), lambda i: (i, 0)),
        out_shape=jax.ShapeDtypeStruct((n, 8), F32),
    )(acc2, t2, b2r)


def kernel(x, edge_index, W1, att_src1, att_dst1, b1, W2, att_src2, att_dst2,
           b2):
    attsf = att_src1.reshape(1, 64)
    attdf = att_dst1.reshape(1, 64)
    asf2 = att_src2.reshape(1, 7)
    adf2 = att_dst2.reshape(1, 7)
    b1r = b1.reshape(1, 64)
    b2r = b2.reshape(1, 7)

    pad = EPAD - E_EDGES
    srcp = jnp.concatenate([edge_index[0], jnp.zeros((pad,), jnp.int32)])
    dstp = jnp.concatenate([edge_index[1], jnp.zeros((pad,), jnp.int32)])

    tsrc, tdst = _tc_a(x, W1, attsf, attdf)
    acc1n, acc1d = _sc_edge1(srcp, dstp, tsrc, tdst)
    t2 = _tc_c(tsrc, tdst, acc1n, acc1d, b1r, W2, asf2, adf2)
    acc2 = _sc_edge2(srcp, dstp, t2)
    outp = _tc_e(acc2, t2, b2r)
    return outp[:, :7]


# speedup vs baseline: 63.2545x; 1.0956x over previous
import jax
import jax.numpy as jnp
from jax import lax
from jax.experimental import pallas as pl
from jax.experimental.pallas import tpu as pltpu
from jax.experimental.pallas import tpu_sc as plsc

F32 = jnp.float32

N_NODES = 50000
E_EDGES = 800000
RB = 400
NC, NS, L = 2, 16, 16
R_HALF = N_NODES // NC
ACC_ROWS = 25088
TRASH = R_HALF
K1 = 64
EPT = 50048
NCH = EPT // K1
EPAD = NS * EPT + 2 * K1
ZROWS = ACC_ROWS // NS

_SC_PARAMS = pltpu.CompilerParams(
    use_tc_tiling_on_sc=False, needs_layout_passes=False)


def _cv(v, dtype=jnp.int32):
    return jnp.full((L,), v, dtype)


def _vsel(vals, idx):
    dn = lax.GatherDimensionNumbers(
        offset_dims=(), collapsed_slice_dims=(0,), start_index_map=(0,))
    return lax.gather(vals, idx.reshape(L, 1), dn, slice_sizes=(1,),
                      mode=lax.GatherScatterMode.PROMISE_IN_BOUNDS)


def _tc_a_body(xt_ref, w_ref, attsf_ref, attdf_ref, tsrc_ref, tdst_ref):
    xw = lax.dot_general(xt_ref[...], w_ref[...],
                         (((0,), (0,)), ((), ())),
                         preferred_element_type=F32)
    r64 = lax.broadcasted_iota(jnp.int32, (64, 8), 0)
    c8 = lax.broadcasted_iota(jnp.int32, (64, 8), 1)
    S = jnp.where(r64 // 8 == c8, 1.0, 0.0).astype(F32)
    asrc = jnp.dot(xw * attsf_ref[...], S, preferred_element_type=F32)
    adst = jnp.dot(xw * attdf_ref[...], S, preferred_element_type=F32)
    tsrc_ref[...] = jnp.concatenate([xw, asrc], axis=1)
    tdst_ref[...] = jnp.concatenate([adst, adst], axis=1)


def _tc_a(xt, W1, attsf, attdf):
    f, n = xt.shape
    rb = 512
    return pl.pallas_call(
        _tc_a_body,
        grid=((n + rb - 1) // rb,),
        in_specs=[pl.BlockSpec((f, rb), lambda i: (0, i)),
                  pl.BlockSpec((f, 64), lambda i: (0, 0)),
                  pl.BlockSpec((1, 64), lambda i: (0, 0)),
                  pl.BlockSpec((1, 64), lambda i: (0, 0))],
        out_specs=[pl.BlockSpec((rb, 72), lambda i: (i, 0)),
                   pl.BlockSpec((rb, 16), lambda i: (i, 0))],
        out_shape=[jax.ShapeDtypeStruct((n, 72), F32),
                   jax.ShapeDtypeStruct((n, 16), F32)],
    )(xt, W1, attsf, attdf)


def _edge_pipeline(ebase, issue_idx, drain_idx, compute_dloc, issue_gather,
                   drain_gather, compute_chunk, issue_scatter, drain_scatter):
    issue_idx(0, ebase)
    drain_idx(0)
    compute_dloc(0, ebase)
    issue_gather(0)
    issue_idx(1, ebase + K1)

    def _pair(k, carry):
        eo0 = ebase + (2 * k) * K1
        drain_gather(0)

        @pl.when(k > 0)
        def _():
            drain_scatter(1)
        drain_idx(1)
        compute_dloc(1, eo0 + K1)
        issue_gather(1)
        issue_idx(0, eo0 + 2 * K1)
        compute_chunk(0)
        issue_scatter(0)
        drain_gather(1)
        drain_scatter(0)
        drain_idx(0)
        compute_dloc(0, eo0 + 2 * K1)
        issue_gather(0)
        issue_idx(1, eo0 + 3 * K1)
        compute_chunk(1)
        issue_scatter(1)
        return carry
    lax.fori_loop(0, NCH // 2, _pair, 0)
    drain_gather(0)
    drain_idx(1)
    drain_scatter(1)


def _sc_edge1_body(src_h, dst_h, tsrc_h, tdst_h, onum_h, oden_h,
                   accn_sh, accd_sh,
                   src0_v, src1_v, dst0_v, dst1_v, dloc0_v, dloc1_v,
                   rows0_v, rows1_v, adst0_v, adst1_v, cnum_v, cden_v,
                   si0, si1, sg0, sg1, ss):
    c = lax.axis_index("c")
    s = lax.axis_index("s")
    lo = c * R_HALF
    lane = lax.iota(jnp.int32, L)
    zv = jnp.zeros((L,), F32)
    lov = jnp.full((L,), lo, jnp.int32)
    zero_i = _cv(0)
    rhalf_v = _cv(R_HALF)
    trash_v = _cv(TRASH)
    eedge_v = _cv(E_EDGES)
    zero_f = _cv(0.0, F32)
    slope_v = _cv(0.2, F32)
    eight_v = _cv(8)
    hrep_idx = [lane // eight_v + _cv(2 * j + 8) for j in range(4)]
    den_mask = lane >= eight_v
    den_col = lane % eight_v
    srcs = (src0_v, src1_v)
    dsts = (dst0_v, dst1_v)
    dlocs = (dloc0_v, dloc1_v)
    rows = (rows0_v, rows1_v)
    adsts = (adst0_v, adst1_v)
    sis = (si0, si1)
    sgs = (sg0, sg1)

    def _zrow(r, carry):
        for j in range(4):
            cnum_v[r, pl.ds(j * L, L)] = zv
        plsc.store_scatter(cden_v, [jnp.full((L,), r, jnp.int32), den_col],
                           zv, mask=den_mask)
        return carry
    lax.fori_loop(0, K1, _zrow, 0)
    zbase = s * ZROWS
    off = 0
    while off < ZROWS:
        sz = min(K1, ZROWS - off)
        pltpu.sync_copy(cnum_v.at[pl.ds(0, sz)],
                        accn_sh.at[pl.ds(zbase + off, sz)])
        pltpu.sync_copy(cden_v.at[pl.ds(0, sz)],
                        accd_sh.at[pl.ds(zbase + off, sz)])
        off += sz
    plsc.subcore_barrier()

    def issue_idx(p, off):
        pltpu.async_copy(src_h.at[pl.ds(off, K1)], srcs[p], sis[p])
        pltpu.async_copy(dst_h.at[pl.ds(off, K1)], dsts[p], sis[p])

    def drain_idx(p):
        pltpu.make_async_copy(src_h.at[pl.ds(0, K1)], srcs[p], sis[p]).wait()
        pltpu.make_async_copy(dst_h.at[pl.ds(0, K1)], dsts[p], sis[p]).wait()

    def compute_dloc(p, eo):
        for i in range(K1 // L):
            d = dsts[p][pl.ds(i * L, L)]
            eid = jnp.full((L,), eo + i * L, jnp.int32) + lane
            dl = d - lov
            inr = (dl >= zero_i) & (dl < rhalf_v) & (eid < eedge_v)
            dlocs[p][pl.ds(i * L, L)] = jnp.where(inr, dl, trash_v)

    def issue_gather(p):
        pltpu.async_copy(tsrc_h.at[srcs[p]], rows[p], sgs[p])
        pltpu.async_copy(tdst_h.at[dsts[p]], adsts[p], sgs[p])

    def drain_gather(p):
        pltpu.make_async_copy(tsrc_h.at[srcs[p]], rows[p], sgs[p]).wait()
        pltpu.make_async_copy(tdst_h.at[dsts[p]], adsts[p], sgs[p]).wait()

    def compute_chunk(p):
        rows_v = rows[p]
        adst_v = adsts[p]

        def _edge4(q, carry2):
            es = [4 * q + u for u in range(4)]
            ws = []
            for e in es:
                asrc = rows_v[e, pl.ds(56, L)]
                ad = adst_v[e, pl.ds(0, L)]
                al = asrc + ad
                al = jnp.where(al >= zero_f, al, al * slope_v)
                ws.append(jnp.exp(al))
            for e, w in zip(es, ws):
                plsc.store_scatter(cden_v,
                                   [jnp.full((L,), e, jnp.int32), den_col],
                                   w, mask=den_mask)
            for j in range(4):
                for e, w in zip(es, ws):
                    wr = _vsel(w, hrep_idx[j])
                    cnum_v[e, pl.ds(j * L, L)] = rows_v[e, pl.ds(j * L, L)] * wr
            return carry2
        lax.fori_loop(0, K1 // 4, _edge4, 0)

    def issue_scatter(p):
        pltpu.async_copy(cnum_v, accn_sh.at[dlocs[p]], ss, add=True)
        pltpu.async_copy(cden_v, accd_sh.at[dlocs[p]], ss, add=True)

    def drain_scatter(p):
        pltpu.make_async_copy(cnum_v, accn_sh.at[dlocs[p]], ss).wait()
        pltpu.make_async_copy(cden_v, accd_sh.at[dlocs[p]], ss).wait()

    _edge_pipeline(s * EPT, issue_idx, drain_idx, compute_dloc, issue_gather,
                   drain_gather, compute_chunk, issue_scatter, drain_scatter)
    plsc.subcore_barrier()

    @pl.when(s < 5)
    def _copy_out():
        nrows = R_HALF // 5
        pltpu.sync_copy(accn_sh.at[pl.ds(s * nrows, nrows)],
                        onum_h.at[pl.ds(c * R_HALF + s * nrows, nrows)])
        pltpu.sync_copy(accd_sh.at[pl.ds(s * nrows, nrows)],
                        oden_h.at[pl.ds(c * R_HALF + s * nrows, nrows)])


def _sc_edge1(srcp, dstp, tsrc, tdst):
    return pl.kernel(
        _sc_edge1_body,
        out_type=[jax.ShapeDtypeStruct((N_NODES, 64), F32),
                  jax.ShapeDtypeStruct((N_NODES, 8), F32)],
        compiler_params=_SC_PARAMS,
        mesh=plsc.VectorSubcoreMesh(core_axis_name="c", subcore_axis_name="s"),
        scratch_types=[
            pltpu.VMEM_SHARED((ACC_ROWS, 64), F32),
            pltpu.VMEM_SHARED((ACC_ROWS, 8), F32),
            pltpu.VMEM((K1,), jnp.int32),
            pltpu.VMEM((K1,), jnp.int32),
            pltpu.VMEM((K1,), jnp.int32),
            pltpu.VMEM((K1,), jnp.int32),
            pltpu.VMEM((K1,), jnp.int32),
            pltpu.VMEM((K1,), jnp.int32),
            pltpu.VMEM((K1, 72), F32),
            pltpu.VMEM((K1, 72), F32),
            pltpu.VMEM((K1, 16), F32),
            pltpu.VMEM((K1, 16), F32),
            pltpu.VMEM((K1, 64), F32),
            pltpu.VMEM((K1, 8), F32),
            pltpu.SemaphoreType.DMA,
            pltpu.SemaphoreType.DMA,
            pltpu.SemaphoreType.DMA,
            pltpu.SemaphoreType.DMA,
            pltpu.SemaphoreType.DMA,
        ],
    )(srcp, dstp, tsrc, tdst)


def _tc_c_body(tsrc_ref, tdst_ref, an_ref, ad_ref, b1_ref, w2_ref, asf_ref,
               adf_ref, t2_ref):
    ts = tsrc_ref[...]
    xw = ts[:, 0:64]
    aw = ts[:, 64:72] + tdst_ref[...][:, 0:8]
    wself = jnp.exp(jnp.where(aw >= 0.0, aw, aw * 0.2))
    inv = 1.0 / (ad_ref[...] + wself + 1e-16)
    r8 = lax.broadcasted_iota(jnp.int32, (8, 64), 0)
    c64 = lax.broadcasted_iota(jnp.int32, (8, 64), 1)
    R8 = jnp.where(r8 == c64 // 8, 1.0, 0.0).astype(F32)
    wrep = jnp.dot(wself, R8, preferred_element_type=F32)
    invrep = jnp.dot(inv, R8, preferred_element_type=F32)
    h1 = (an_ref[...] + wrep * xw) * invrep + b1_ref[...]
    h1 = jnp.where(h1 > 0.0, h1, jnp.exp(jnp.minimum(h1, 0.0)) - 1.0)
    xw2 = jnp.dot(h1, w2_ref[...], preferred_element_type=F32)
    asrc2 = jnp.sum(xw2 * asf_ref[...], axis=1, keepdims=True)
    adst2 = jnp.sum(xw2 * adf_ref[...], axis=1, keepdims=True)
    one1 = jnp.ones((xw2.shape[0], 1), F32)
    z6 = jnp.zeros((xw2.shape[0], 6), F32)
    t2_ref[...] = jnp.concatenate([xw2, one1, asrc2, adst2, z6], axis=1)


def _tc_c(tsrc, tdst, acc1n, acc1d, b1r, W2, asf2, adf2):
    n = tsrc.shape[0]
    return pl.pallas_call(
        _tc_c_body,
        grid=(n // RB,),
        in_specs=[pl.BlockSpec((RB, 72), lambda i: (i, 0)),
                  pl.BlockSpec((RB, 16), lambda i: (i, 0)),
                  pl.BlockSpec((RB, 64), lambda i: (i, 0)),
                  pl.BlockSpec((RB, 8), lambda i: (i, 0)),
                  pl.BlockSpec((1, 64), lambda i: (0, 0)),
                  pl.BlockSpec((64, 7), lambda i: (0, 0)),
                  pl.BlockSpec((1, 7), lambda i: (0, 0)),
                  pl.BlockSpec((1, 7), lambda i: (0, 0))],
        out_specs=pl.BlockSpec((RB, 16), lambda i: (i, 0)),
        out_shape=jax.ShapeDtypeStruct((n, 16), F32),
    )(tsrc, tdst, acc1n, acc1d, b1r, W2, asf2, adf2)


def _sc_edge2_body(src_h, dst_h, t2_h, out_h,
                   acc_sh,
                   src0_v, src1_v, dst0_v, dst1_v, dloc0_v, dloc1_v,
                   rows0_v, rows1_v, adst0_v, adst1_v, con_v,
                   si0, si1, sg0, sg1, ss):
    c = lax.axis_index("c")
    s = lax.axis_index("s")
    lo = c * R_HALF
    lane = lax.iota(jnp.int32, L)
    zv = jnp.zeros((L,), F32)
    lov = jnp.full((L,), lo, jnp.int32)
    zero_i = _cv(0)
    rhalf_v = _cv(R_HALF)
    trash_v = _cv(TRASH)
    eedge_v = _cv(E_EDGES)
    zero_f = _cv(0.0, F32)
    slope_v = _cv(0.2, F32)
    eight_v = _cv(8)
    nine_v = _cv(9)
    srcs = (src0_v, src1_v)
    dsts = (dst0_v, dst1_v)
    dlocs = (dloc0_v, dloc1_v)
    rows = (rows0_v, rows1_v)
    adsts = (adst0_v, adst1_v)
    sis = (si0, si1)
    sgs = (sg0, sg1)

    def _zrow(r, carry):
        con_v[r, pl.ds(0, L)] = zv
        return carry
    lax.fori_loop(0, K1, _zrow, 0)
    zbase = s * ZROWS
    off = 0
    while off < ZROWS:
        sz = min(K1, ZROWS - off)
        pltpu.sync_copy(con_v.at[pl.ds(0, sz)],
                        acc_sh.at[pl.ds(zbase + off, sz)])
        off += sz
    plsc.subcore_barrier()

    def issue_idx(p, off):
        pltpu.async_copy(src_h.at[pl.ds(off, K1)], srcs[p], sis[p])
        pltpu.async_copy(dst_h.at[pl.ds(off, K1)], dsts[p], sis[p])

    def drain_idx(p):
        pltpu.make_async_copy(src_h.at[pl.ds(0, K1)], srcs[p], sis[p]).wait()
        pltpu.make_async_copy(dst_h.at[pl.ds(0, K1)], dsts[p], sis[p]).wait()

    def compute_dloc(p, eo):
        for i in range(K1 // L):
            d = dsts[p][pl.ds(i * L, L)]
            eid = jnp.full((L,), eo + i * L, jnp.int32) + lane
            dl = d - lov
            inr = (dl >= zero_i) & (dl < rhalf_v) & (eid < eedge_v)
            dlocs[p][pl.ds(i * L, L)] = jnp.where(inr, dl, trash_v)

    def issue_gather(p):
        pltpu.async_copy(t2_h.at[srcs[p]], rows[p], sgs[p])
        pltpu.async_copy(t2_h.at[dsts[p]], adsts[p], sgs[p])

    def drain_gather(p):
        pltpu.make_async_copy(t2_h.at[srcs[p]], rows[p], sgs[p]).wait()
        pltpu.make_async_copy(t2_h.at[dsts[p]], adsts[p], sgs[p]).wait()

    def compute_chunk(p):
        rows_v = rows[p]
        adst_v = adsts[p]

        def _grp(q, carry2):
            base = q * L
            ev = jnp.full((L,), base, jnp.int32) + lane
            a1 = plsc.load_gather(rows_v, [ev, eight_v])
            a2 = plsc.load_gather(adst_v, [ev, nine_v])
            al = a1 + a2
            al = jnp.where(al >= zero_f, al, al * slope_v)
            w16 = jnp.exp(al)
            for u in range(L):
                e = base + u
                wspl = _vsel(w16, _cv(u))
                con_v[e, pl.ds(0, L)] = rows_v[e, pl.ds(0, L)] * wspl
            return carry2
        lax.fori_loop(0, K1 // L, _grp, 0)

    def issue_scatter(p):
        pltpu.async_copy(con_v, acc_sh.at[dlocs[p]], ss, add=True)

    def drain_scatter(p):
        pltpu.make_async_copy(con_v, acc_sh.at[dlocs[p]], ss).wait()

    _edge_pipeline(s * EPT, issue_idx, drain_idx, compute_dloc, issue_gather,
                   drain_gather, compute_chunk, issue_scatter, drain_scatter)
    plsc.subcore_barrier()

    @pl.when(s < 5)
    def _copy_out():
        nrows = R_HALF // 5
        pltpu.sync_copy(acc_sh.at[pl.ds(s * nrows, nrows)],
                        out_h.at[pl.ds(c * R_HALF + s * nrows, nrows)])


def _sc_edge2(srcp, dstp, t2):
    return pl.kernel(
        _sc_edge2_body,
        out_type=jax.ShapeDtypeStruct((N_NODES, 16), F32),
        compiler_params=_SC_PARAMS,
        mesh=plsc.VectorSubcoreMesh(core_axis_name="c", subcore_axis_name="s"),
        scratch_types=[
            pltpu.VMEM_SHARED((ACC_ROWS, 16), F32),
            pltpu.VMEM((K1,), jnp.int32),
            pltpu.VMEM((K1,), jnp.int32),
            pltpu.VMEM((K1,), jnp.int32),
            pltpu.VMEM((K1,), jnp.int32),
            pltpu.VMEM((K1,), jnp.int32),
            pltpu.VMEM((K1,), jnp.int32),
            pltpu.VMEM((K1, 16), F32),
            pltpu.VMEM((K1, 16), F32),
            pltpu.VMEM((K1, 16), F32),
            pltpu.VMEM((K1, 16), F32),
            pltpu.VMEM((K1, 16), F32),
            pltpu.SemaphoreType.DMA,
            pltpu.SemaphoreType.DMA,
            pltpu.SemaphoreType.DMA,
            pltpu.SemaphoreType.DMA,
            pltpu.SemaphoreType.DMA,
        ],
    )(srcp, dstp, t2)


def _tc_e_body(acc_ref, t2_ref, b2_ref, out_ref):
    ac = acc_ref[...]
    t2 = t2_ref[...]
    a2 = t2[:, 8:9] + t2[:, 9:10]
    w = jnp.exp(jnp.where(a2 >= 0.0, a2, a2 * 0.2))
    num = ac[:, 0:7] + w * t2[:, 0:7]
    den = ac[:, 7:8] + w + 1e-16
    o = num / den + b2_ref[...]
    m = jnp.max(o, axis=1, keepdims=True)
    sh = o - m
    lse = jnp.log(jnp.sum(jnp.exp(sh), axis=1, keepdims=True))
    res = sh - lse
    out_ref[...] = jnp.concatenate(
        [res, jnp.zeros((res.shape[0], 1), F32)], axis=1)


def _tc_e(acc2, t2, b2r):
    n = acc2.shape[0]
    return pl.pallas_call(
        _tc_e_body,
        grid=(n // RB,),
        in_specs=[pl.BlockSpec((RB, 16), lambda i: (i, 0)),
                  pl.BlockSpec((RB, 16), lambda i: (i, 0)),
                  pl.BlockSpec((1, 7), lambda i: (0, 0))],
        out_specs=pl.BlockSpec((RB, 8), lambda i: (i, 0)),
        out_shape=jax.ShapeDtypeStruct((n, 8), F32),
    )(acc2, t2, b2r)


def kernel(x, edge_index, W1, att_src1, att_dst1, b1, W2, att_src2, att_dst2,
           b2):
    attsf = att_src1.reshape(1, 64)
    attdf = att_dst1.reshape(1, 64)
    asf2 = att_src2.reshape(1, 7)
    adf2 = att_dst2.reshape(1, 7)
    b1r = b1.reshape(1, 64)
    b2r = b2.reshape(1, 7)

    pad = EPAD - E_EDGES
    srcp = jnp.concatenate([edge_index[0], jnp.zeros((pad,), jnp.int32)])
    dstp = jnp.concatenate([edge_index[1], jnp.zeros((pad,), jnp.int32)])

    tsrc, tdst = _tc_a(x.T, W1, attsf, attdf)
    acc1n, acc1d = _sc_edge1(srcp, dstp, tsrc, tdst)
    t2 = _tc_c(tsrc, tdst, acc1n, acc1d, b1r, W2, asf2, adf2)
    acc2 = _sc_edge2(srcp, dstp, t2)
    outp = _tc_e(acc2, t2, b2r)
    return outp[:, :7]


# speedup vs baseline: 65.5003x; 1.0355x over previous
import jax
import jax.numpy as jnp
from jax import lax
from jax.experimental import pallas as pl
from jax.experimental.pallas import tpu as pltpu
from jax.experimental.pallas import tpu_sc as plsc

F32 = jnp.float32

N_NODES = 50000
E_EDGES = 800000
RB = 400
NC, NS, L = 2, 16, 16
R_HALF = N_NODES // NC
ACC_ROWS = 25008
TRASH = R_HALF
K1 = 64
K2 = 128
EPT = 50176
EPAD = NS * EPT + 256
ZROWS = ACC_ROWS // NS

_SC_PARAMS = pltpu.CompilerParams(
    use_tc_tiling_on_sc=False, needs_layout_passes=False)


def _cv(v, dtype=jnp.int32):
    return jnp.full((L,), v, dtype)


def _vsel(vals, idx):
    dn = lax.GatherDimensionNumbers(
        offset_dims=(), collapsed_slice_dims=(0,), start_index_map=(0,))
    return lax.gather(vals, idx.reshape(L, 1), dn, slice_sizes=(1,),
                      mode=lax.GatherScatterMode.PROMISE_IN_BOUNDS)


def _tc_a_body(xt_ref, w_ref, attsf_ref, attdf_ref, tsrc_ref, tdst_ref):
    xw = lax.dot_general(xt_ref[...], w_ref[...],
                         (((0,), (0,)), ((), ())),
                         preferred_element_type=F32)
    r64 = lax.broadcasted_iota(jnp.int32, (64, 8), 0)
    c8 = lax.broadcasted_iota(jnp.int32, (64, 8), 1)
    S = jnp.where(r64 // 8 == c8, 1.0, 0.0).astype(F32)
    asrc = jnp.dot(xw * attsf_ref[...], S, preferred_element_type=F32)
    adst = jnp.dot(xw * attdf_ref[...], S, preferred_element_type=F32)
    tsrc_ref[...] = jnp.concatenate([xw, asrc], axis=1)
    tdst_ref[...] = jnp.concatenate([adst, adst], axis=1)


def _tc_a(xt, W1, attsf, attdf):
    f, n = xt.shape
    rb = 512
    return pl.pallas_call(
        _tc_a_body,
        grid=((n + rb - 1) // rb,),
        in_specs=[pl.BlockSpec((f, rb), lambda i: (0, i)),
                  pl.BlockSpec((f, 64), lambda i: (0, 0)),
                  pl.BlockSpec((1, 64), lambda i: (0, 0)),
                  pl.BlockSpec((1, 64), lambda i: (0, 0))],
        out_specs=[pl.BlockSpec((rb, 72), lambda i: (i, 0)),
                   pl.BlockSpec((rb, 16), lambda i: (i, 0))],
        out_shape=[jax.ShapeDtypeStruct((n, 72), F32),
                   jax.ShapeDtypeStruct((n, 16), F32)],
    )(xt, W1, attsf, attdf)


def _edge_pipeline(ebase, K, nch, lag, issue_idx, drain_idx, compute_dloc,
                   issue_gather, drain_gather, compute_chunk, issue_scatter,
                   drain_scatter):
    issue_idx(0, ebase)
    drain_idx(0)
    compute_dloc(0, 0, ebase)
    issue_gather(0)
    issue_idx(1, ebase + K)

    def _quad(k, carry):
        for u in range(4):
            p = u % 2
            eo = ebase + (4 * k + u) * K
            drain_idx(1 - p)
            compute_dloc(1 - p, (u + 1) % 4, eo + K)
            issue_gather(1 - p)
            drain_gather(p)
            issue_idx(p, eo + 2 * K)
            if u >= lag:
                drain_scatter((u - lag) % 4, (u - lag) % lag if lag > 1 else 0)
            else:
                @pl.when(k > 0)
                def _():
                    drain_scatter((u - lag) % 4, (u - lag) % lag if lag > 1 else 0)
            compute_chunk(p, u % lag if lag > 1 else 0)
            issue_scatter(u % 4, u % lag if lag > 1 else 0)
        return carry
    lax.fori_loop(0, nch // 4, _quad, 0)
    drain_gather(0)
    drain_idx(1)
    for i in range(lag):
        g = nch - lag + i
        drain_scatter(g % 4, g % lag if lag > 1 else 0)


def _sc_edge1_body(src_h, dst_h, tsrc_h, tdst_h, out_h,
                   acc_sh,
                   src0_v, src1_v, dst0_v, dst1_v,
                   dl0_v, dl1_v, dl2_v, dl3_v,
                   rows0_v, rows1_v, adst0_v, adst1_v, con_v,
                   si0, si1, sg0, sg1, ss):
    c = lax.axis_index("c")
    s = lax.axis_index("s")
    lo = c * R_HALF
    lane = lax.iota(jnp.int32, L)
    zv = jnp.zeros((L,), F32)
    lov = jnp.full((L,), lo, jnp.int32)
    zero_i = _cv(0)
    rhalf_v = _cv(R_HALF)
    trash_v = _cv(TRASH)
    eedge_v = _cv(E_EDGES)
    zero_f = _cv(0.0, F32)
    slope_v = _cv(0.2, F32)
    eight_v = _cv(8)
    hrep_idx = [lane // eight_v + _cv(2 * j + 8) for j in range(4)]
    den_mask = lane >= eight_v
    den_col = lane % eight_v + _cv(64)
    srcs = (src0_v, src1_v)
    dsts = (dst0_v, dst1_v)
    dlocs = (dl0_v, dl1_v, dl2_v, dl3_v)
    rows = (rows0_v, rows1_v)
    adsts = (adst0_v, adst1_v)
    sis = (si0, si1)
    sgs = (sg0, sg1)

    def _zrow(r, carry):
        for j in range(4):
            con_v[r, pl.ds(j * L, L)] = zv
        plsc.store_scatter(con_v, [jnp.full((L,), r, jnp.int32), den_col],
                           zv, mask=den_mask)
        return carry
    lax.fori_loop(0, K1, _zrow, 0)
    zbase = s * ZROWS
    off = 0
    while off < ZROWS:
        sz = min(K1, ZROWS - off)
        pltpu.sync_copy(con_v.at[pl.ds(0, sz)],
                        acc_sh.at[pl.ds(zbase + off, sz)])
        off += sz
    plsc.subcore_barrier()

    def issue_idx(p, off):
        pltpu.async_copy(src_h.at[pl.ds(off, K1)], srcs[p], sis[p])
        pltpu.async_copy(dst_h.at[pl.ds(off, K1)], dsts[p], sis[p])

    def drain_idx(p):
        pltpu.make_async_copy(src_h.at[pl.ds(0, K1)], srcs[p], sis[p]).wait()
        pltpu.make_async_copy(dst_h.at[pl.ds(0, K1)], dsts[p], sis[p]).wait()

    def compute_dloc(p, slot, eo):
        for i in range(K1 // L):
            d = dsts[p][pl.ds(i * L, L)]
            eid = jnp.full((L,), eo + i * L, jnp.int32) + lane
            dl = d - lov
            inr = (dl >= zero_i) & (dl < rhalf_v) & (eid < eedge_v)
            dlocs[slot][pl.ds(i * L, L)] = jnp.where(inr, dl, trash_v)

    def issue_gather(p):
        pltpu.async_copy(tsrc_h.at[srcs[p]], rows[p], sgs[p])
        pltpu.async_copy(tdst_h.at[dsts[p]], adsts[p], sgs[p])

    def drain_gather(p):
        pltpu.make_async_copy(tsrc_h.at[srcs[p]], rows[p], sgs[p]).wait()
        pltpu.make_async_copy(tdst_h.at[dsts[p]], adsts[p], sgs[p]).wait()

    def compute_chunk(p, conp):
        rows_v = rows[p]
        adst_v = adsts[p]

        def _edge4(q, carry2):
            es = [4 * q + u for u in range(4)]
            ws = []
            for e in es:
                asrc = rows_v[e, pl.ds(56, L)]
                ad = adst_v[e, pl.ds(0, L)]
                al = asrc + ad
                al = jnp.where(al >= zero_f, al, al * slope_v)
                ws.append(jnp.exp(al))
            for e, w in zip(es, ws):
                plsc.store_scatter(con_v,
                                   [jnp.full((L,), e, jnp.int32), den_col],
                                   w, mask=den_mask)
            for j in range(4):
                for e, w in zip(es, ws):
                    wr = _vsel(w, hrep_idx[j])
                    con_v[e, pl.ds(j * L, L)] = rows_v[e, pl.ds(j * L, L)] * wr
            return carry2
        lax.fori_loop(0, K1 // 4, _edge4, 0)

    def issue_scatter(slot, conp):
        pltpu.async_copy(con_v, acc_sh.at[dlocs[slot]], ss, add=True)

    def drain_scatter(slot, conp):
        pltpu.make_async_copy(con_v, acc_sh.at[dlocs[slot]], ss).wait()

    _edge_pipeline(s * EPT, K1, EPT // K1, 1,
                   issue_idx, drain_idx, compute_dloc, issue_gather,
                   drain_gather, compute_chunk, issue_scatter, drain_scatter)
    plsc.subcore_barrier()

    @pl.when(s < 5)
    def _copy_out():
        nrows = R_HALF // 5
        pltpu.sync_copy(acc_sh.at[pl.ds(s * nrows, nrows)],
                        out_h.at[pl.ds(c * R_HALF + s * nrows, nrows)])


def _sc_edge1(srcp, dstp, tsrc, tdst):
    return pl.kernel(
        _sc_edge1_body,
        out_type=jax.ShapeDtypeStruct((N_NODES, 72), F32),
        compiler_params=_SC_PARAMS,
        mesh=plsc.VectorSubcoreMesh(core_axis_name="c", subcore_axis_name="s"),
        scratch_types=[
            pltpu.VMEM_SHARED((ACC_ROWS, 72), F32),
            pltpu.VMEM((K1,), jnp.int32),
            pltpu.VMEM((K1,), jnp.int32),
            pltpu.VMEM((K1,), jnp.int32),
            pltpu.VMEM((K1,), jnp.int32),
            pltpu.VMEM((K1,), jnp.int32),
            pltpu.VMEM((K1,), jnp.int32),
            pltpu.VMEM((K1,), jnp.int32),
            pltpu.VMEM((K1,), jnp.int32),
            pltpu.VMEM((K1, 72), F32),
            pltpu.VMEM((K1, 72), F32),
            pltpu.VMEM((K1, 16), F32),
            pltpu.VMEM((K1, 16), F32),
            pltpu.VMEM((K1, 72), F32),
            pltpu.SemaphoreType.DMA,
            pltpu.SemaphoreType.DMA,
            pltpu.SemaphoreType.DMA,
            pltpu.SemaphoreType.DMA,
            pltpu.SemaphoreType.DMA,
        ],
    )(srcp, dstp, tsrc, tdst)


def _tc_c_body(tsrc_ref, tdst_ref, acc_ref, b1_ref, w2_ref, asf_ref,
               adf_ref, t2_ref):
    ts = tsrc_ref[...]
    ac = acc_ref[...]
    xw = ts[:, 0:64]
    aw = ts[:, 64:72] + tdst_ref[...][:, 0:8]
    wself = jnp.exp(jnp.where(aw >= 0.0, aw, aw * 0.2))
    inv = 1.0 / (ac[:, 64:72] + wself + 1e-16)
    r8 = lax.broadcasted_iota(jnp.int32, (8, 64), 0)
    c64 = lax.broadcasted_iota(jnp.int32, (8, 64), 1)
    R8 = jnp.where(r8 == c64 // 8, 1.0, 0.0).astype(F32)
    wrep = jnp.dot(wself, R8, preferred_element_type=F32)
    invrep = jnp.dot(inv, R8, preferred_element_type=F32)
    h1 = (ac[:, 0:64] + wrep * xw) * invrep + b1_ref[...]
    h1 = jnp.where(h1 > 0.0, h1, jnp.exp(jnp.minimum(h1, 0.0)) - 1.0)
    xw2 = jnp.dot(h1, w2_ref[...], preferred_element_type=F32)
    asrc2 = jnp.sum(xw2 * asf_ref[...], axis=1, keepdims=True)
    adst2 = jnp.sum(xw2 * adf_ref[...], axis=1, keepdims=True)
    one1 = jnp.ones((xw2.shape[0], 1), F32)
    z6 = jnp.zeros((xw2.shape[0], 6), F32)
    t2_ref[...] = jnp.concatenate([xw2, one1, asrc2, adst2, z6], axis=1)


def _tc_c(tsrc, tdst, acc1, b1r, W2, asf2, adf2):
    n = tsrc.shape[0]
    return pl.pallas_call(
        _tc_c_body,
        grid=(n // RB,),
        in_specs=[pl.BlockSpec((RB, 72), lambda i: (i, 0)),
                  pl.BlockSpec((RB, 16), lambda i: (i, 0)),
                  pl.BlockSpec((RB, 72), lambda i: (i, 0)),
                  pl.BlockSpec((1, 64), lambda i: (0, 0)),
                  pl.BlockSpec((64, 7), lambda i: (0, 0)),
                  pl.BlockSpec((1, 7), lambda i: (0, 0)),
                  pl.BlockSpec((1, 7), lambda i: (0, 0))],
        out_specs=pl.BlockSpec((RB, 16), lambda i: (i, 0)),
        out_shape=jax.ShapeDtypeStruct((n, 16), F32),
    )(tsrc, tdst, acc1, b1r, W2, asf2, adf2)


def _sc_edge2_body(src_h, dst_h, t2_h, out_h,
                   acc_sh,
                   src0_v, src1_v, dst0_v, dst1_v,
                   dl0_v, dl1_v, dl2_v, dl3_v,
                   rows0_v, rows1_v, adst0_v, adst1_v, con0_v, con1_v,
                   si0, si1, sg0, sg1, ss):
    c = lax.axis_index("c")
    s = lax.axis_index("s")
    lo = c * R_HALF
    lane = lax.iota(jnp.int32, L)
    zv = jnp.zeros((L,), F32)
    lov = jnp.full((L,), lo, jnp.int32)
    zero_i = _cv(0)
    rhalf_v = _cv(R_HALF)
    trash_v = _cv(TRASH)
    eedge_v = _cv(E_EDGES)
    zero_f = _cv(0.0, F32)
    slope_v = _cv(0.2, F32)
    eight_v = _cv(8)
    nine_v = _cv(9)
    srcs = (src0_v, src1_v)
    dsts = (dst0_v, dst1_v)
    dlocs = (dl0_v, dl1_v, dl2_v, dl3_v)
    rows = (rows0_v, rows1_v)
    adsts = (adst0_v, adst1_v)
    cons = (con0_v, con1_v)
    sis = (si0, si1)
    sgs = (sg0, sg1)

    def _zrow(r, carry):
        con0_v[r, pl.ds(0, L)] = zv
        return carry
    lax.fori_loop(0, K2, _zrow, 0)
    zbase = s * ZROWS
    off = 0
    while off < ZROWS:
        sz = min(K2, ZROWS - off)
        pltpu.sync_copy(con0_v.at[pl.ds(0, sz)],
                        acc_sh.at[pl.ds(zbase + off, sz)])
        off += sz
    plsc.subcore_barrier()

    def issue_idx(p, off):
        pltpu.async_copy(src_h.at[pl.ds(off, K2)], srcs[p], sis[p])
        pltpu.async_copy(dst_h.at[pl.ds(off, K2)], dsts[p], sis[p])

    def drain_idx(p):
        pltpu.make_async_copy(src_h.at[pl.ds(0, K2)], srcs[p], sis[p]).wait()
        pltpu.make_async_copy(dst_h.at[pl.ds(0, K2)], dsts[p], sis[p]).wait()

    def compute_dloc(p, slot, eo):
        for i in range(K2 // L):
            d = dsts[p][pl.ds(i * L, L)]
            eid = jnp.full((L,), eo + i * L, jnp.int32) + lane
            dl = d - lov
            inr = (dl >= zero_i) & (dl < rhalf_v) & (eid < eedge_v)
            dlocs[slot][pl.ds(i * L, L)] = jnp.where(inr, dl, trash_v)

    def issue_gather(p):
        pltpu.async_copy(t2_h.at[srcs[p]], rows[p], sgs[p])
        pltpu.async_copy(t2_h.at[dsts[p]], adsts[p], sgs[p])

    def drain_gather(p):
        pltpu.make_async_copy(t2_h.at[srcs[p]], rows[p], sgs[p]).wait()
        pltpu.make_async_copy(t2_h.at[dsts[p]], adsts[p], sgs[p]).wait()

    def compute_chunk(p, conp):
        rows_v = rows[p]
        adst_v = adsts[p]
        con_v = cons[conp]

        def _grp(q, carry2):
            base = q * L
            ev = jnp.full((L,), base, jnp.int32) + lane
            a1 = plsc.load_gather(rows_v, [ev, eight_v])
            a2 = plsc.load_gather(adst_v, [ev, nine_v])
            al = a1 + a2
            al = jnp.where(al >= zero_f, al, al * slope_v)
            w16 = jnp.exp(al)
            for u in range(L):
                e = base + u
                wspl = _vsel(w16, _cv(u))
                con_v[e, pl.ds(0, L)] = rows_v[e, pl.ds(0, L)] * wspl
            return carry2
        lax.fori_loop(0, K2 // L, _grp, 0)

    def issue_scatter(slot, conp):
        pltpu.async_copy(cons[conp], acc_sh.at[dlocs[slot]], ss, add=True)

    def drain_scatter(slot, conp):
        pltpu.make_async_copy(cons[conp], acc_sh.at[dlocs[slot]], ss).wait()

    _edge_pipeline(s * EPT, K2, EPT // K2, 2,
                   issue_idx, drain_idx, compute_dloc, issue_gather,
                   drain_gather, compute_chunk, issue_scatter, drain_scatter)
    plsc.subcore_barrier()

    @pl.when(s < 5)
    def _copy_out():
        nrows = R_HALF // 5
        pltpu.sync_copy(acc_sh.at[pl.ds(s * nrows, nrows)],
                        out_h.at[pl.ds(c * R_HALF + s * nrows, nrows)])


def _sc_edge2(srcp, dstp, t2):
    return pl.kernel(
        _sc_edge2_body,
        out_type=jax.ShapeDtypeStruct((N_NODES, 16), F32),
        compiler_params=_SC_PARAMS,
        mesh=plsc.VectorSubcoreMesh(core_axis_name="c", subcore_axis_name="s"),
        scratch_types=[
            pltpu.VMEM_SHARED((ACC_ROWS, 16), F32),
            pltpu.VMEM((K2,), jnp.int32),
            pltpu.VMEM((K2,), jnp.int32),
            pltpu.VMEM((K2,), jnp.int32),
            pltpu.VMEM((K2,), jnp.int32),
            pltpu.VMEM((K2,), jnp.int32),
            pltpu.VMEM((K2,), jnp.int32),
            pltpu.VMEM((K2,), jnp.int32),
            pltpu.VMEM((K2,), jnp.int32),
            pltpu.VMEM((K2, 16), F32),
            pltpu.VMEM((K2, 16), F32),
            pltpu.VMEM((K2, 16), F32),
            pltpu.VMEM((K2, 16), F32),
            pltpu.VMEM((K2, 16), F32),
            pltpu.VMEM((K2, 16), F32),
            pltpu.SemaphoreType.DMA,
            pltpu.SemaphoreType.DMA,
            pltpu.SemaphoreType.DMA,
            pltpu.SemaphoreType.DMA,
            pltpu.SemaphoreType.DMA,
        ],
    )(srcp, dstp, t2)


def _tc_e_body(acc_ref, t2_ref, b2_ref, out_ref):
    ac = acc_ref[...]
    t2 = t2_ref[...]
    a2 = t2[:, 8:9] + t2[:, 9:10]
    w = jnp.exp(jnp.where(a2 >= 0.0, a2, a2 * 0.2))
    num = ac[:, 0:7] + w * t2[:, 0:7]
    den = ac[:, 7:8] + w + 1e-16
    o = num / den + b2_ref[...]
    m = jnp.max(o, axis=1, keepdims=True)
    sh = o - m
    lse = jnp.log(jnp.sum(jnp.exp(sh), axis=1, keepdims=True))
    res = sh - lse
    out_ref[...] = jnp.concatenate(
        [res, jnp.zeros((res.shape[0], 1), F32)], axis=1)


def _tc_e(acc2, t2, b2r):
    n = acc2.shape[0]
    return pl.pallas_call(
        _tc_e_body,
        grid=(n // RB,),
        in_specs=[pl.BlockSpec((RB, 16), lambda i: (i, 0)),
                  pl.BlockSpec((RB, 16), lambda i: (i, 0)),
                  pl.BlockSpec((1, 7), lambda i: (0, 0))],
        out_specs=pl.BlockSpec((RB, 8), lambda i: (i, 0)),
        out_shape=jax.ShapeDtypeStruct((n, 8), F32),
    )(acc2, t2, b2r)


def kernel(x, edge_index, W1, att_src1, att_dst1, b1, W2, att_src2, att_dst2,
           b2):
    attsf = att_src1.reshape(1, 64)
    attdf = att_dst1.reshape(1, 64)
    asf2 = att_src2.reshape(1, 7)
    adf2 = att_dst2.reshape(1, 7)
    b1r = b1.reshape(1, 64)
    b2r = b2.reshape(1, 7)

    pad = EPAD - E_EDGES
    srcp = jnp.concatenate([edge_index[0], jnp.zeros((pad,), jnp.int32)])
    dstp = jnp.concatenate([edge_index[1], jnp.zeros((pad,), jnp.int32)])

    tsrc, tdst = _tc_a(x.T, W1, attsf, attdf)
    acc1 = _sc_edge1(srcp, dstp, tsrc, tdst)
    t2 = _tc_c(tsrc, tdst, acc1, b1r, W2, asf2, adf2)
    acc2 = _sc_edge2(srcp, dstp, t2)
    outp = _tc_e(acc2, t2, b2r)
    return outp[:, :7]


# speedup vs baseline: 66.2115x; 1.0109x over previous
import jax
import jax.numpy as jnp
from jax import lax
from jax.experimental import pallas as pl
from jax.experimental.pallas import tpu as pltpu
from jax.experimental.pallas import tpu_sc as plsc

F32 = jnp.float32

N_NODES = 50000
E_EDGES = 800000
RB = 400
NC, NS, L = 2, 16, 16
R_HALF = N_NODES // NC
ACC_ROWS = 25008
TRASH = R_HALF
K1 = 64
K2 = 128
EPT = 50176
EPAD = NS * EPT + 256
ZROWS = ACC_ROWS // NS

_SC_PARAMS = pltpu.CompilerParams(
    use_tc_tiling_on_sc=False, needs_layout_passes=False)


def _cv(v, dtype=jnp.int32):
    return jnp.full((L,), v, dtype)


def _vsel(vals, idx):
    dn = lax.GatherDimensionNumbers(
        offset_dims=(), collapsed_slice_dims=(0,), start_index_map=(0,))
    return lax.gather(vals, idx.reshape(L, 1), dn, slice_sizes=(1,),
                      mode=lax.GatherScatterMode.PROMISE_IN_BOUNDS)


def _tc_a_body(xt_ref, w_ref, attsf_ref, attdf_ref, tsrc_ref, tdst_ref):
    xw = lax.dot_general(xt_ref[...], w_ref[...],
                         (((0,), (0,)), ((), ())),
                         preferred_element_type=F32)
    r64 = lax.broadcasted_iota(jnp.int32, (64, 8), 0)
    c8 = lax.broadcasted_iota(jnp.int32, (64, 8), 1)
    S = jnp.where(r64 // 8 == c8, 1.0, 0.0).astype(F32)
    asrc = jnp.dot(xw * attsf_ref[...], S, preferred_element_type=F32)
    adst = jnp.dot(xw * attdf_ref[...], S, preferred_element_type=F32)
    tsrc_ref[...] = jnp.concatenate([xw, asrc], axis=1)
    tdst_ref[...] = jnp.concatenate([adst, adst], axis=1)


def _tc_a(xt, W1, attsf, attdf):
    f, n = xt.shape
    rb = 512
    return pl.pallas_call(
        _tc_a_body,
        grid=((n + rb - 1) // rb,),
        in_specs=[pl.BlockSpec((f, rb), lambda i: (0, i)),
                  pl.BlockSpec((f, 64), lambda i: (0, 0)),
                  pl.BlockSpec((1, 64), lambda i: (0, 0)),
                  pl.BlockSpec((1, 64), lambda i: (0, 0))],
        out_specs=[pl.BlockSpec((rb, 72), lambda i: (i, 0)),
                   pl.BlockSpec((rb, 16), lambda i: (i, 0))],
        out_shape=[jax.ShapeDtypeStruct((n, 72), F32),
                   jax.ShapeDtypeStruct((n, 16), F32)],
    )(xt, W1, attsf, attdf)


def _edge_pipeline(ebase, K, nch, lag, issue_idx, drain_idx, compute_dloc,
                   issue_gather, drain_gather, compute_chunk, issue_scatter,
                   drain_scatter):
    issue_idx(0, ebase)
    drain_idx(0)
    compute_dloc(0, 0, ebase)
    issue_gather(0)
    issue_idx(1, ebase + K)

    def _quad(k, carry):
        for u in range(4):
            p = u % 2
            eo = ebase + (4 * k + u) * K
            drain_idx(1 - p)
            compute_dloc(1 - p, (u + 1) % 4, eo + K)
            issue_gather(1 - p)
            drain_gather(p)
            issue_idx(p, eo + 2 * K)
            if u >= lag:
                drain_scatter((u - lag) % 4, (u - lag) % lag if lag > 1 else 0)
            else:
                @pl.when(k > 0)
                def _():
                    drain_scatter((u - lag) % 4, (u - lag) % lag if lag > 1 else 0)
            compute_chunk(p, u % lag if lag > 1 else 0)
            issue_scatter(u % 4, u % lag if lag > 1 else 0)
        return carry
    lax.fori_loop(0, nch // 4, _quad, 0)
    drain_gather(0)
    drain_idx(1)
    for i in range(lag):
        g = nch - lag + i
        drain_scatter(g % 4, g % lag if lag > 1 else 0)


def _sc_edge1_body(src_h, dst_h, tsrc_h, tdst_h, out_h,
                   acc_sh,
                   src0_v, src1_v, dst0_v, dst1_v,
                   dl0_v, dl1_v, dl2_v, dl3_v,
                   rows0_v, rows1_v, adst0_v, adst1_v, con_v,
                   si0, si1, sg0, sg1, ss):
    c = lax.axis_index("c")
    s = lax.axis_index("s")
    lo = c * R_HALF
    lane = lax.iota(jnp.int32, L)
    zv = jnp.zeros((L,), F32)
    lov = jnp.full((L,), lo, jnp.int32)
    zero_i = _cv(0)
    rhalf_v = _cv(R_HALF)
    trash_v = _cv(TRASH)
    eedge_v = _cv(E_EDGES)
    zero_f = _cv(0.0, F32)
    slope_v = _cv(0.2, F32)
    eight_v = _cv(8)
    hrep_idx = [lane // eight_v + _cv(2 * j + 8) for j in range(4)]
    den_mask = lane >= eight_v
    den_col = lane % eight_v + _cv(64)
    srcs = (src0_v, src1_v)
    dsts = (dst0_v, dst1_v)
    dlocs = (dl0_v, dl1_v, dl2_v, dl3_v)
    rows = (rows0_v, rows1_v)
    adsts = (adst0_v, adst1_v)
    sis = (si0, si1)
    sgs = (sg0, sg1)

    def _zrow(r, carry):
        for j in range(4):
            con_v[r, pl.ds(j * L, L)] = zv
        plsc.store_scatter(con_v, [jnp.full((L,), r, jnp.int32), den_col],
                           zv, mask=den_mask)
        return carry
    lax.fori_loop(0, K1, _zrow, 0)
    zbase = s * ZROWS
    off = 0
    while off < ZROWS:
        sz = min(K1, ZROWS - off)
        pltpu.sync_copy(con_v.at[pl.ds(0, sz)],
                        acc_sh.at[pl.ds(zbase + off, sz)])
        off += sz
    plsc.subcore_barrier()

    def issue_idx(p, off):
        pltpu.async_copy(src_h.at[pl.ds(off, K1)], srcs[p], sis[p])
        pltpu.async_copy(dst_h.at[pl.ds(off, K1)], dsts[p], sis[p])

    def drain_idx(p):
        pltpu.make_async_copy(src_h.at[pl.ds(0, K1)], srcs[p], sis[p]).wait()
        pltpu.make_async_copy(dst_h.at[pl.ds(0, K1)], dsts[p], sis[p]).wait()

    def compute_dloc(p, slot, eo):
        for i in range(K1 // L):
            d = dsts[p][pl.ds(i * L, L)]
            eid = jnp.full((L,), eo + i * L, jnp.int32) + lane
            dl = d - lov
            inr = (dl >= zero_i) & (dl < rhalf_v) & (eid < eedge_v)
            dlocs[slot][pl.ds(i * L, L)] = jnp.where(inr, dl, trash_v)

    def issue_gather(p):
        pltpu.async_copy(tsrc_h.at[srcs[p]], rows[p], sgs[p])
        pltpu.async_copy(tdst_h.at[dsts[p]], adsts[p], sgs[p])

    def drain_gather(p):
        pltpu.make_async_copy(tsrc_h.at[srcs[p]], rows[p], sgs[p]).wait()
        pltpu.make_async_copy(tdst_h.at[dsts[p]], adsts[p], sgs[p]).wait()

    def compute_chunk(p, conp):
        rows_v = rows[p]
        adst_v = adsts[p]

        def _edge4(q, carry2):
            es = [8 * q + u for u in range(8)]
            ws = []
            for e in es:
                asrc = rows_v[e, pl.ds(56, L)]
                ad = adst_v[e, pl.ds(0, L)]
                al = asrc + ad
                al = jnp.where(al >= zero_f, al, al * slope_v)
                ws.append(jnp.exp(al))
            for e, w in zip(es, ws):
                plsc.store_scatter(con_v,
                                   [jnp.full((L,), e, jnp.int32), den_col],
                                   w, mask=den_mask)
            for j in range(4):
                for e, w in zip(es, ws):
                    wr = _vsel(w, hrep_idx[j])
                    con_v[e, pl.ds(j * L, L)] = rows_v[e, pl.ds(j * L, L)] * wr
            return carry2
        lax.fori_loop(0, K1 // 8, _edge4, 0)

    def issue_scatter(slot, conp):
        pltpu.async_copy(con_v, acc_sh.at[dlocs[slot]], ss, add=True)

    def drain_scatter(slot, conp):
        pltpu.make_async_copy(con_v, acc_sh.at[dlocs[slot]], ss).wait()

    _edge_pipeline(s * EPT, K1, EPT // K1, 1,
                   issue_idx, drain_idx, compute_dloc, issue_gather,
                   drain_gather, compute_chunk, issue_scatter, drain_scatter)
    plsc.subcore_barrier()

    @pl.when(s < 5)
    def _copy_out():
        nrows = R_HALF // 5
        pltpu.sync_copy(acc_sh.at[pl.ds(s * nrows, nrows)],
                        out_h.at[pl.ds(c * R_HALF + s * nrows, nrows)])


def _sc_edge1(srcp, dstp, tsrc, tdst):
    return pl.kernel(
        _sc_edge1_body,
        out_type=jax.ShapeDtypeStruct((N_NODES, 72), F32),
        compiler_params=_SC_PARAMS,
        mesh=plsc.VectorSubcoreMesh(core_axis_name="c", subcore_axis_name="s"),
        scratch_types=[
            pltpu.VMEM_SHARED((ACC_ROWS, 72), F32),
            pltpu.VMEM((K1,), jnp.int32),
            pltpu.VMEM((K1,), jnp.int32),
            pltpu.VMEM((K1,), jnp.int32),
            pltpu.VMEM((K1,), jnp.int32),
            pltpu.VMEM((K1,), jnp.int32),
            pltpu.VMEM((K1,), jnp.int32),
            pltpu.VMEM((K1,), jnp.int32),
            pltpu.VMEM((K1,), jnp.int32),
            pltpu.VMEM((K1, 72), F32),
            pltpu.VMEM((K1, 72), F32),
            pltpu.VMEM((K1, 16), F32),
            pltpu.VMEM((K1, 16), F32),
            pltpu.VMEM((K1, 72), F32),
            pltpu.SemaphoreType.DMA,
            pltpu.SemaphoreType.DMA,
            pltpu.SemaphoreType.DMA,
            pltpu.SemaphoreType.DMA,
            pltpu.SemaphoreType.DMA,
        ],
    )(srcp, dstp, tsrc, tdst)


def _tc_c_body(tsrc_ref, tdst_ref, acc_ref, b1_ref, w2_ref, asf_ref,
               adf_ref, t2_ref):
    ts = tsrc_ref[...]
    ac = acc_ref[...]
    xw = ts[:, 0:64]
    aw = ts[:, 64:72] + tdst_ref[...][:, 0:8]
    wself = jnp.exp(jnp.where(aw >= 0.0, aw, aw * 0.2))
    inv = 1.0 / (ac[:, 64:72] + wself + 1e-16)
    r8 = lax.broadcasted_iota(jnp.int32, (8, 64), 0)
    c64 = lax.broadcasted_iota(jnp.int32, (8, 64), 1)
    R8 = jnp.where(r8 == c64 // 8, 1.0, 0.0).astype(F32)
    wrep = jnp.dot(wself, R8, preferred_element_type=F32)
    invrep = jnp.dot(inv, R8, preferred_element_type=F32)
    h1 = (ac[:, 0:64] + wrep * xw) * invrep + b1_ref[...]
    h1 = jnp.where(h1 > 0.0, h1, jnp.exp(jnp.minimum(h1, 0.0)) - 1.0)
    xw2 = jnp.dot(h1, w2_ref[...], preferred_element_type=F32)
    asrc2 = jnp.sum(xw2 * asf_ref[...], axis=1, keepdims=True)
    adst2 = jnp.sum(xw2 * adf_ref[...], axis=1, keepdims=True)
    one1 = jnp.ones((xw2.shape[0], 1), F32)
    z6 = jnp.zeros((xw2.shape[0], 6), F32)
    t2_ref[...] = jnp.concatenate([xw2, one1, asrc2, adst2, z6], axis=1)


def _tc_c(tsrc, tdst, acc1, b1r, W2, asf2, adf2):
    n = tsrc.shape[0]
    return pl.pallas_call(
        _tc_c_body,
        grid=(n // RB,),
        in_specs=[pl.BlockSpec((RB, 72), lambda i: (i, 0)),
                  pl.BlockSpec((RB, 16), lambda i: (i, 0)),
                  pl.BlockSpec((RB, 72), lambda i: (i, 0)),
                  pl.BlockSpec((1, 64), lambda i: (0, 0)),
                  pl.BlockSpec((64, 7), lambda i: (0, 0)),
                  pl.BlockSpec((1, 7), lambda i: (0, 0)),
                  pl.BlockSpec((1, 7), lambda i: (0, 0))],
        out_specs=pl.BlockSpec((RB, 16), lambda i: (i, 0)),
        out_shape=jax.ShapeDtypeStruct((n, 16), F32),
    )(tsrc, tdst, acc1, b1r, W2, asf2, adf2)


def _sc_edge2_body(src_h, dst_h, t2_h, out_h,
                   acc_sh,
                   src0_v, src1_v, dst0_v, dst1_v,
                   dl0_v, dl1_v, dl2_v, dl3_v,
                   rows0_v, rows1_v, adst0_v, adst1_v, con0_v, con1_v,
                   si0, si1, sg0, sg1, ss):
    c = lax.axis_index("c")
    s = lax.axis_index("s")
    lo = c * R_HALF
    lane = lax.iota(jnp.int32, L)
    zv = jnp.zeros((L,), F32)
    lov = jnp.full((L,), lo, jnp.int32)
    zero_i = _cv(0)
    rhalf_v = _cv(R_HALF)
    trash_v = _cv(TRASH)
    eedge_v = _cv(E_EDGES)
    zero_f = _cv(0.0, F32)
    slope_v = _cv(0.2, F32)
    eight_v = _cv(8)
    nine_v = _cv(9)
    srcs = (src0_v, src1_v)
    dsts = (dst0_v, dst1_v)
    dlocs = (dl0_v, dl1_v, dl2_v, dl3_v)
    rows = (rows0_v, rows1_v)
    adsts = (adst0_v, adst1_v)
    cons = (con0_v, con1_v)
    sis = (si0, si1)
    sgs = (sg0, sg1)

    def _zrow(r, carry):
        con0_v[r, pl.ds(0, L)] = zv
        return carry
    lax.fori_loop(0, K2, _zrow, 0)
    zbase = s * ZROWS
    off = 0
    while off < ZROWS:
        sz = min(K2, ZROWS - off)
        pltpu.sync_copy(con0_v.at[pl.ds(0, sz)],
                        acc_sh.at[pl.ds(zbase + off, sz)])
        off += sz
    plsc.subcore_barrier()

    def issue_idx(p, off):
        pltpu.async_copy(src_h.at[pl.ds(off, K2)], srcs[p], sis[p])
        pltpu.async_copy(dst_h.at[pl.ds(off, K2)], dsts[p], sis[p])

    def drain_idx(p):
        pltpu.make_async_copy(src_h.at[pl.ds(0, K2)], srcs[p], sis[p]).wait()
        pltpu.make_async_copy(dst_h.at[pl.ds(0, K2)], dsts[p], sis[p]).wait()

    def compute_dloc(p, slot, eo):
        for i in range(K2 // L):
            d = dsts[p][pl.ds(i * L, L)]
            eid = jnp.full((L,), eo + i * L, jnp.int32) + lane
            dl = d - lov
            inr = (dl >= zero_i) & (dl < rhalf_v) & (eid < eedge_v)
            dlocs[slot][pl.ds(i * L, L)] = jnp.where(inr, dl, trash_v)

    def issue_gather(p):
        pltpu.async_copy(t2_h.at[srcs[p]], rows[p], sgs[p])
        pltpu.async_copy(t2_h.at[dsts[p]], adsts[p], sgs[p])

    def drain_gather(p):
        pltpu.make_async_copy(t2_h.at[srcs[p]], rows[p], sgs[p]).wait()
        pltpu.make_async_copy(t2_h.at[dsts[p]], adsts[p], sgs[p]).wait()

    def compute_chunk(p, conp):
        rows_v = rows[p]
        adst_v = adsts[p]
        con_v = cons[conp]

        def _grp(q, carry2):
            base = q * L
            ev = jnp.full((L,), base, jnp.int32) + lane
            a1 = plsc.load_gather(rows_v, [ev, eight_v])
            a2 = plsc.load_gather(adst_v, [ev, nine_v])
            al = a1 + a2
            al = jnp.where(al >= zero_f, al, al * slope_v)
            w16 = jnp.exp(al)
            for u in range(L):
                e = base + u
                wspl = _vsel(w16, _cv(u))
                con_v[e, pl.ds(0, L)] = rows_v[e, pl.ds(0, L)] * wspl
            return carry2
        lax.fori_loop(0, K2 // L, _grp, 0)

    def issue_scatter(slot, conp):
        pltpu.async_copy(cons[conp], acc_sh.at[dlocs[slot]], ss, add=True)

    def drain_scatter(slot, conp):
        pltpu.make_async_copy(cons[conp], acc_sh.at[dlocs[slot]], ss).wait()

    _edge_pipeline(s * EPT, K2, EPT // K2, 2,
                   issue_idx, drain_idx, compute_dloc, issue_gather,
                   drain_gather, compute_chunk, issue_scatter, drain_scatter)
    plsc.subcore_barrier()

    @pl.when(s < 5)
    def _copy_out():
        nrows = R_HALF // 5
        pltpu.sync_copy(acc_sh.at[pl.ds(s * nrows, nrows)],
                        out_h.at[pl.ds(c * R_HALF + s * nrows, nrows)])


def _sc_edge2(srcp, dstp, t2):
    return pl.kernel(
        _sc_edge2_body,
        out_type=jax.ShapeDtypeStruct((N_NODES, 16), F32),
        compiler_params=_SC_PARAMS,
        mesh=plsc.VectorSubcoreMesh(core_axis_name="c", subcore_axis_name="s"),
        scratch_types=[
            pltpu.VMEM_SHARED((ACC_ROWS, 16), F32),
            pltpu.VMEM((K2,), jnp.int32),
            pltpu.VMEM((K2,), jnp.int32),
            pltpu.VMEM((K2,), jnp.int32),
            pltpu.VMEM((K2,), jnp.int32),
            pltpu.VMEM((K2,), jnp.int32),
            pltpu.VMEM((K2,), jnp.int32),
            pltpu.VMEM((K2,), jnp.int32),
            pltpu.VMEM((K2,), jnp.int32),
            pltpu.VMEM((K2, 16), F32),
            pltpu.VMEM((K2, 16), F32),
            pltpu.VMEM((K2, 16), F32),
            pltpu.VMEM((K2, 16), F32),
            pltpu.VMEM((K2, 16), F32),
            pltpu.VMEM((K2, 16), F32),
            pltpu.SemaphoreType.DMA,
            pltpu.SemaphoreType.DMA,
            pltpu.SemaphoreType.DMA,
            pltpu.SemaphoreType.DMA,
            pltpu.SemaphoreType.DMA,
        ],
    )(srcp, dstp, t2)


def _tc_e_body(acc_ref, t2_ref, b2_ref, out_ref):
    ac = acc_ref[...]
    t2 = t2_ref[...]
    a2 = t2[:, 8:9] + t2[:, 9:10]
    w = jnp.exp(jnp.where(a2 >= 0.0, a2, a2 * 0.2))
    num = ac[:, 0:7] + w * t2[:, 0:7]
    den = ac[:, 7:8] + w + 1e-16
    o = num / den + b2_ref[...]
    m = jnp.max(o, axis=1, keepdims=True)
    sh = o - m
    lse = jnp.log(jnp.sum(jnp.exp(sh), axis=1, keepdims=True))
    res = sh - lse
    out_ref[...] = jnp.concatenate(
        [res, jnp.zeros((res.shape[0], 1), F32)], axis=1)


def _tc_e(acc2, t2, b2r):
    n = acc2.shape[0]
    return pl.pallas_call(
        _tc_e_body,
        grid=(n // RB,),
        in_specs=[pl.BlockSpec((RB, 16), lambda i: (i, 0)),
                  pl.BlockSpec((RB, 16), lambda i: (i, 0)),
                  pl.BlockSpec((1, 7), lambda i: (0, 0))],
        out_specs=pl.BlockSpec((RB, 8), lambda i: (i, 0)),
        out_shape=jax.ShapeDtypeStruct((n, 8), F32),
    )(acc2, t2, b2r)


def kernel(x, edge_index, W1, att_src1, att_dst1, b1, W2, att_src2, att_dst2,
           b2):
    attsf = att_src1.reshape(1, 64)
    attdf = att_dst1.reshape(1, 64)
    asf2 = att_src2.reshape(1, 7)
    adf2 = att_dst2.reshape(1, 7)
    b1r = b1.reshape(1, 64)
    b2r = b2.reshape(1, 7)

    pad = EPAD - E_EDGES
    srcp = jnp.concatenate([edge_index[0], jnp.zeros((pad,), jnp.int32)])
    dstp = jnp.concatenate([edge_index[1], jnp.zeros((pad,), jnp.int32)])

    tsrc, tdst = _tc_a(x.T, W1, attsf, attdf)
    acc1 = _sc_edge1(srcp, dstp, tsrc, tdst)
    t2 = _tc_c(tsrc, tdst, acc1, b1r, W2, asf2, adf2)
    acc2 = _sc_edge2(srcp, dstp, t2)
    outp = _tc_e(acc2, t2, b2r)
    return outp[:, :7]


# speedup vs baseline: 67.5238x; 1.0198x over previous
import jax
import jax.numpy as jnp
from jax import lax
from jax.experimental import pallas as pl
from jax.experimental.pallas import tpu as pltpu
from jax.experimental.pallas import tpu_sc as plsc

F32 = jnp.float32

N_NODES = 50000
E_EDGES = 800000
RB = 400
NC, NS, L = 2, 16, 16
R_HALF = N_NODES // NC
ACC_ROWS = 25008
TRASH = R_HALF
K1 = 64
K2 = 128
EPT = 50176
EPAD = NS * EPT + 256
ZROWS = ACC_ROWS // NS

_SC_PARAMS = pltpu.CompilerParams(
    use_tc_tiling_on_sc=False, needs_layout_passes=False)


def _cv(v, dtype=jnp.int32):
    return jnp.full((L,), v, dtype)


def _vsel(vals, idx):
    dn = lax.GatherDimensionNumbers(
        offset_dims=(), collapsed_slice_dims=(0,), start_index_map=(0,))
    return lax.gather(vals, idx.reshape(L, 1), dn, slice_sizes=(1,),
                      mode=lax.GatherScatterMode.PROMISE_IN_BOUNDS)


def _tc_a_body(xt_ref, w_ref, attsf_ref, attdf_ref, tsrc_ref, tdst_ref):
    xw = lax.dot_general(xt_ref[...], w_ref[...],
                         (((0,), (0,)), ((), ())),
                         preferred_element_type=F32)
    r64 = lax.broadcasted_iota(jnp.int32, (64, 8), 0)
    c8 = lax.broadcasted_iota(jnp.int32, (64, 8), 1)
    S = jnp.where(r64 % 8 == c8, 1.0, 0.0).astype(F32)
    asrc = jnp.dot(xw * attsf_ref[...], S, preferred_element_type=F32)
    adst = jnp.dot(xw * attdf_ref[...], S, preferred_element_type=F32)
    tsrc_ref[...] = jnp.concatenate([xw, asrc], axis=1)
    tdst_ref[...] = jnp.concatenate([adst, adst], axis=1)


def _tc_a(xt, W1, attsf, attdf):
    f, n = xt.shape
    rb = 512
    return pl.pallas_call(
        _tc_a_body,
        grid=((n + rb - 1) // rb,),
        in_specs=[pl.BlockSpec((f, rb), lambda i: (0, i)),
                  pl.BlockSpec((f, 64), lambda i: (0, 0)),
                  pl.BlockSpec((1, 64), lambda i: (0, 0)),
                  pl.BlockSpec((1, 64), lambda i: (0, 0))],
        out_specs=[pl.BlockSpec((rb, 72), lambda i: (i, 0)),
                   pl.BlockSpec((rb, 16), lambda i: (i, 0))],
        out_shape=[jax.ShapeDtypeStruct((n, 72), F32),
                   jax.ShapeDtypeStruct((n, 16), F32)],
    )(xt, W1, attsf, attdf)


def _edge_pipeline(ebase, K, nch, lag, issue_idx, drain_idx, compute_dloc,
                   issue_gather, drain_gather, compute_chunk, issue_scatter,
                   drain_scatter):
    issue_idx(0, ebase)
    drain_idx(0)
    compute_dloc(0, 0, ebase)
    issue_gather(0)
    issue_idx(1, ebase + K)

    def _quad(k, carry):
        for u in range(4):
            p = u % 2
            eo = ebase + (4 * k + u) * K
            drain_idx(1 - p)
            compute_dloc(1 - p, (u + 1) % 4, eo + K)
            issue_gather(1 - p)
            drain_gather(p)
            issue_idx(p, eo + 2 * K)
            if u >= lag:
                drain_scatter((u - lag) % 4, (u - lag) % lag if lag > 1 else 0)
            else:
                @pl.when(k > 0)
                def _():
                    drain_scatter((u - lag) % 4, (u - lag) % lag if lag > 1 else 0)
            compute_chunk(p, u % lag if lag > 1 else 0)
            issue_scatter(u % 4, u % lag if lag > 1 else 0)
        return carry
    lax.fori_loop(0, nch // 4, _quad, 0)
    drain_gather(0)
    drain_idx(1)
    for i in range(lag):
        g = nch - lag + i
        drain_scatter(g % 4, g % lag if lag > 1 else 0)


def _sc_edge1_body(src_h, dst_h, tsrc_h, tdst_h, out_h,
                   acc_sh,
                   src0_v, src1_v, dst0_v, dst1_v,
                   dl0_v, dl1_v, dl2_v, dl3_v,
                   rows0_v, rows1_v, adst0_v, adst1_v, con_v,
                   si0, si1, sg0, sg1, ss):
    c = lax.axis_index("c")
    s = lax.axis_index("s")
    lo = c * R_HALF
    lane = lax.iota(jnp.int32, L)
    zv = jnp.zeros((L,), F32)
    lov = jnp.full((L,), lo, jnp.int32)
    zero_i = _cv(0)
    rhalf_v = _cv(R_HALF)
    trash_v = _cv(TRASH)
    eedge_v = _cv(E_EDGES)
    zero_f = _cv(0.0, F32)
    slope_v = _cv(0.2, F32)
    eight_v = _cv(8)
    wrep_idx = lane % eight_v + eight_v
    den_mask = lane >= eight_v
    den_col = lane % eight_v + _cv(64)
    srcs = (src0_v, src1_v)
    dsts = (dst0_v, dst1_v)
    dlocs = (dl0_v, dl1_v, dl2_v, dl3_v)
    rows = (rows0_v, rows1_v)
    adsts = (adst0_v, adst1_v)
    sis = (si0, si1)
    sgs = (sg0, sg1)

    def _zrow(r, carry):
        for j in range(4):
            con_v[r, pl.ds(j * L, L)] = zv
        plsc.store_scatter(con_v, [jnp.full((L,), r, jnp.int32), den_col],
                           zv, mask=den_mask)
        return carry
    lax.fori_loop(0, K1, _zrow, 0)
    zbase = s * ZROWS
    off = 0
    while off < ZROWS:
        sz = min(K1, ZROWS - off)
        pltpu.sync_copy(con_v.at[pl.ds(0, sz)],
                        acc_sh.at[pl.ds(zbase + off, sz)])
        off += sz
    plsc.subcore_barrier()

    def issue_idx(p, off):
        pltpu.async_copy(src_h.at[pl.ds(off, K1)], srcs[p], sis[p])
        pltpu.async_copy(dst_h.at[pl.ds(off, K1)], dsts[p], sis[p])

    def drain_idx(p):
        pltpu.make_async_copy(src_h.at[pl.ds(0, K1)], srcs[p], sis[p]).wait()
        pltpu.make_async_copy(dst_h.at[pl.ds(0, K1)], dsts[p], sis[p]).wait()

    def compute_dloc(p, slot, eo):
        for i in range(K1 // L):
            d = dsts[p][pl.ds(i * L, L)]
            eid = jnp.full((L,), eo + i * L, jnp.int32) + lane
            dl = d - lov
            inr = (dl >= zero_i) & (dl < rhalf_v) & (eid < eedge_v)
            dlocs[slot][pl.ds(i * L, L)] = jnp.where(inr, dl, trash_v)

    def issue_gather(p):
        pltpu.async_copy(tsrc_h.at[srcs[p]], rows[p], sgs[p])
        pltpu.async_copy(tdst_h.at[dsts[p]], adsts[p], sgs[p])

    def drain_gather(p):
        pltpu.make_async_copy(tsrc_h.at[srcs[p]], rows[p], sgs[p]).wait()
        pltpu.make_async_copy(tdst_h.at[dsts[p]], adsts[p], sgs[p]).wait()

    def compute_chunk(p, conp):
        rows_v = rows[p]
        adst_v = adsts[p]

        def _edge4(q, carry2):
            es = [8 * q + u for u in range(8)]
            ws = []
            for e in es:
                asrc = rows_v[e, pl.ds(56, L)]
                ad = adst_v[e, pl.ds(0, L)]
                al = asrc + ad
                al = jnp.where(al >= zero_f, al, al * slope_v)
                ws.append(jnp.exp(al))
            for e, w in zip(es, ws):
                plsc.store_scatter(con_v,
                                   [jnp.full((L,), e, jnp.int32), den_col],
                                   w, mask=den_mask)
            wrs = [_vsel(w, wrep_idx) for w in ws]
            for j in range(4):
                for e, wr in zip(es, wrs):
                    con_v[e, pl.ds(j * L, L)] = rows_v[e, pl.ds(j * L, L)] * wr
            return carry2
        lax.fori_loop(0, K1 // 8, _edge4, 0)

    def issue_scatter(slot, conp):
        pltpu.async_copy(con_v, acc_sh.at[dlocs[slot]], ss, add=True)

    def drain_scatter(slot, conp):
        pltpu.make_async_copy(con_v, acc_sh.at[dlocs[slot]], ss).wait()

    _edge_pipeline(s * EPT, K1, EPT // K1, 1,
                   issue_idx, drain_idx, compute_dloc, issue_gather,
                   drain_gather, compute_chunk, issue_scatter, drain_scatter)
    plsc.subcore_barrier()

    @pl.when(s < 5)
    def _copy_out():
        nrows = R_HALF // 5
        pltpu.sync_copy(acc_sh.at[pl.ds(s * nrows, nrows)],
                        out_h.at[pl.ds(c * R_HALF + s * nrows, nrows)])


def _sc_edge1(srcp, dstp, tsrc, tdst):
    return pl.kernel(
        _sc_edge1_body,
        out_type=jax.ShapeDtypeStruct((N_NODES, 72), F32),
        compiler_params=_SC_PARAMS,
        mesh=plsc.VectorSubcoreMesh(core_axis_name="c", subcore_axis_name="s"),
        scratch_types=[
            pltpu.VMEM_SHARED((ACC_ROWS, 72), F32),
            pltpu.VMEM((K1,), jnp.int32),
            pltpu.VMEM((K1,), jnp.int32),
            pltpu.VMEM((K1,), jnp.int32),
            pltpu.VMEM((K1,), jnp.int32),
            pltpu.VMEM((K1,), jnp.int32),
            pltpu.VMEM((K1,), jnp.int32),
            pltpu.VMEM((K1,), jnp.int32),
            pltpu.VMEM((K1,), jnp.int32),
            pltpu.VMEM((K1, 72), F32),
            pltpu.VMEM((K1, 72), F32),
            pltpu.VMEM((K1, 16), F32),
            pltpu.VMEM((K1, 16), F32),
            pltpu.VMEM((K1, 72), F32),
            pltpu.SemaphoreType.DMA,
            pltpu.SemaphoreType.DMA,
            pltpu.SemaphoreType.DMA,
            pltpu.SemaphoreType.DMA,
            pltpu.SemaphoreType.DMA,
        ],
    )(srcp, dstp, tsrc, tdst)


def _tc_c_body(tsrc_ref, tdst_ref, acc_ref, b1_ref, w2_ref, asf_ref,
               adf_ref, t2_ref):
    ts = tsrc_ref[...]
    ac = acc_ref[...]
    xw = ts[:, 0:64]
    aw = ts[:, 64:72] + tdst_ref[...][:, 0:8]
    wself = jnp.exp(jnp.where(aw >= 0.0, aw, aw * 0.2))
    inv = 1.0 / (ac[:, 64:72] + wself + 1e-16)
    r8 = lax.broadcasted_iota(jnp.int32, (8, 64), 0)
    c64 = lax.broadcasted_iota(jnp.int32, (8, 64), 1)
    R8 = jnp.where(r8 == c64 % 8, 1.0, 0.0).astype(F32)
    wrep = jnp.dot(wself, R8, preferred_element_type=F32)
    invrep = jnp.dot(inv, R8, preferred_element_type=F32)
    h1 = (ac[:, 0:64] + wrep * xw) * invrep + b1_ref[...]
    h1 = jnp.where(h1 > 0.0, h1, jnp.exp(jnp.minimum(h1, 0.0)) - 1.0)
    xw2 = jnp.dot(h1, w2_ref[...], preferred_element_type=F32)
    asrc2 = jnp.sum(xw2 * asf_ref[...], axis=1, keepdims=True)
    adst2 = jnp.sum(xw2 * adf_ref[...], axis=1, keepdims=True)
    one1 = jnp.ones((xw2.shape[0], 1), F32)
    z6 = jnp.zeros((xw2.shape[0], 6), F32)
    t2_ref[...] = jnp.concatenate([xw2, one1, asrc2, adst2, z6], axis=1)


def _tc_c(tsrc, tdst, acc1, b1r, W2, asf2, adf2):
    n = tsrc.shape[0]
    return pl.pallas_call(
        _tc_c_body,
        grid=(n // RB,),
        in_specs=[pl.BlockSpec((RB, 72), lambda i: (i, 0)),
                  pl.BlockSpec((RB, 16), lambda i: (i, 0)),
                  pl.BlockSpec((RB, 72), lambda i: (i, 0)),
                  pl.BlockSpec((1, 64), lambda i: (0, 0)),
                  pl.BlockSpec((64, 7), lambda i: (0, 0)),
                  pl.BlockSpec((1, 7), lambda i: (0, 0)),
                  pl.BlockSpec((1, 7), lambda i: (0, 0))],
        out_specs=pl.BlockSpec((RB, 16), lambda i: (i, 0)),
        out_shape=jax.ShapeDtypeStruct((n, 16), F32),
    )(tsrc, tdst, acc1, b1r, W2, asf2, adf2)


def _sc_edge2_body(src_h, dst_h, t2_h, out_h,
                   acc_sh,
                   src0_v, src1_v, dst0_v, dst1_v,
                   dl0_v, dl1_v, dl2_v, dl3_v,
                   rows0_v, rows1_v, adst0_v, adst1_v, con0_v, con1_v,
                   si0, si1, sg0, sg1, ss):
    c = lax.axis_index("c")
    s = lax.axis_index("s")
    lo = c * R_HALF
    lane = lax.iota(jnp.int32, L)
    zv = jnp.zeros((L,), F32)
    lov = jnp.full((L,), lo, jnp.int32)
    zero_i = _cv(0)
    rhalf_v = _cv(R_HALF)
    trash_v = _cv(TRASH)
    eedge_v = _cv(E_EDGES)
    zero_f = _cv(0.0, F32)
    slope_v = _cv(0.2, F32)
    eight_v = _cv(8)
    nine_v = _cv(9)
    srcs = (src0_v, src1_v)
    dsts = (dst0_v, dst1_v)
    dlocs = (dl0_v, dl1_v, dl2_v, dl3_v)
    rows = (rows0_v, rows1_v)
    adsts = (adst0_v, adst1_v)
    cons = (con0_v, con1_v)
    sis = (si0, si1)
    sgs = (sg0, sg1)

    def _zrow(r, carry):
        con0_v[r, pl.ds(0, L)] = zv
        return carry
    lax.fori_loop(0, K2, _zrow, 0)
    zbase = s * ZROWS
    off = 0
    while off < ZROWS:
        sz = min(K2, ZROWS - off)
        pltpu.sync_copy(con0_v.at[pl.ds(0, sz)],
                        acc_sh.at[pl.ds(zbase + off, sz)])
        off += sz
    plsc.subcore_barrier()

    def issue_idx(p, off):
        pltpu.async_copy(src_h.at[pl.ds(off, K2)], srcs[p], sis[p])
        pltpu.async_copy(dst_h.at[pl.ds(off, K2)], dsts[p], sis[p])

    def drain_idx(p):
        pltpu.make_async_copy(src_h.at[pl.ds(0, K2)], srcs[p], sis[p]).wait()
        pltpu.make_async_copy(dst_h.at[pl.ds(0, K2)], dsts[p], sis[p]).wait()

    def compute_dloc(p, slot, eo):
        for i in range(K2 // L):
            d = dsts[p][pl.ds(i * L, L)]
            eid = jnp.full((L,), eo + i * L, jnp.int32) + lane
            dl = d - lov
            inr = (dl >= zero_i) & (dl < rhalf_v) & (eid < eedge_v)
            dlocs[slot][pl.ds(i * L, L)] = jnp.where(inr, dl, trash_v)

    def issue_gather(p):
        pltpu.async_copy(t2_h.at[srcs[p]], rows[p], sgs[p])
        pltpu.async_copy(t2_h.at[dsts[p]], adsts[p], sgs[p])

    def drain_gather(p):
        pltpu.make_async_copy(t2_h.at[srcs[p]], rows[p], sgs[p]).wait()
        pltpu.make_async_copy(t2_h.at[dsts[p]], adsts[p], sgs[p]).wait()

    def compute_chunk(p, conp):
        rows_v = rows[p]
        adst_v = adsts[p]
        con_v = cons[conp]

        def _grp(q, carry2):
            base = q * L
            ev = jnp.full((L,), base, jnp.int32) + lane
            a1 = plsc.load_gather(rows_v, [ev, eight_v])
            a2 = plsc.load_gather(adst_v, [ev, nine_v])
            al = a1 + a2
            al = jnp.where(al >= zero_f, al, al * slope_v)
            w16 = jnp.exp(al)
            for u in range(L):
                e = base + u
                wspl = _vsel(w16, _cv(u))
                con_v[e, pl.ds(0, L)] = rows_v[e, pl.ds(0, L)] * wspl
            return carry2
        lax.fori_loop(0, K2 // L, _grp, 0)

    def issue_scatter(slot, conp):
        pltpu.async_copy(cons[conp], acc_sh.at[dlocs[slot]], ss, add=True)

    def drain_scatter(slot, conp):
        pltpu.make_async_copy(cons[conp], acc_sh.at[dlocs[slot]], ss).wait()

    _edge_pipeline(s * EPT, K2, EPT // K2, 2,
                   issue_idx, drain_idx, compute_dloc, issue_gather,
                   drain_gather, compute_chunk, issue_scatter, drain_scatter)
    plsc.subcore_barrier()

    @pl.when(s < 5)
    def _copy_out():
        nrows = R_HALF // 5
        pltpu.sync_copy(acc_sh.at[pl.ds(s * nrows, nrows)],
                        out_h.at[pl.ds(c * R_HALF + s * nrows, nrows)])


def _sc_edge2(srcp, dstp, t2):
    return pl.kernel(
        _sc_edge2_body,
        out_type=jax.ShapeDtypeStruct((N_NODES, 16), F32),
        compiler_params=_SC_PARAMS,
        mesh=plsc.VectorSubcoreMesh(core_axis_name="c", subcore_axis_name="s"),
        scratch_types=[
            pltpu.VMEM_SHARED((ACC_ROWS, 16), F32),
            pltpu.VMEM((K2,), jnp.int32),
            pltpu.VMEM((K2,), jnp.int32),
            pltpu.VMEM((K2,), jnp.int32),
            pltpu.VMEM((K2,), jnp.int32),
            pltpu.VMEM((K2,), jnp.int32),
            pltpu.VMEM((K2,), jnp.int32),
            pltpu.VMEM((K2,), jnp.int32),
            pltpu.VMEM((K2,), jnp.int32),
            pltpu.VMEM((K2, 16), F32),
            pltpu.VMEM((K2, 16), F32),
            pltpu.VMEM((K2, 16), F32),
            pltpu.VMEM((K2, 16), F32),
            pltpu.VMEM((K2, 16), F32),
            pltpu.VMEM((K2, 16), F32),
            pltpu.SemaphoreType.DMA,
            pltpu.SemaphoreType.DMA,
            pltpu.SemaphoreType.DMA,
            pltpu.SemaphoreType.DMA,
            pltpu.SemaphoreType.DMA,
        ],
    )(srcp, dstp, t2)


def _tc_e_body(acc_ref, t2_ref, b2_ref, out_ref):
    ac = acc_ref[...]
    t2 = t2_ref[...]
    a2 = t2[:, 8:9] + t2[:, 9:10]
    w = jnp.exp(jnp.where(a2 >= 0.0, a2, a2 * 0.2))
    num = ac[:, 0:7] + w * t2[:, 0:7]
    den = ac[:, 7:8] + w + 1e-16
    o = num / den + b2_ref[...]
    m = jnp.max(o, axis=1, keepdims=True)
    sh = o - m
    lse = jnp.log(jnp.sum(jnp.exp(sh), axis=1, keepdims=True))
    res = sh - lse
    out_ref[...] = jnp.concatenate(
        [res, jnp.zeros((res.shape[0], 1), F32)], axis=1)


def _tc_e(acc2, t2, b2r):
    n = acc2.shape[0]
    return pl.pallas_call(
        _tc_e_body,
        grid=(n // RB,),
        in_specs=[pl.BlockSpec((RB, 16), lambda i: (i, 0)),
                  pl.BlockSpec((RB, 16), lambda i: (i, 0)),
                  pl.BlockSpec((1, 7), lambda i: (0, 0))],
        out_specs=pl.BlockSpec((RB, 8), lambda i: (i, 0)),
        out_shape=jax.ShapeDtypeStruct((n, 8), F32),
    )(acc2, t2, b2r)


def kernel(x, edge_index, W1, att_src1, att_dst1, b1, W2, att_src2, att_dst2,
           b2):
    W1cm = W1.reshape(-1, 8, 8).transpose(0, 2, 1).reshape(-1, 64)
    attsf = att_src1.transpose(0, 2, 1).reshape(1, 64)
    attdf = att_dst1.transpose(0, 2, 1).reshape(1, 64)
    b1r = b1.reshape(8, 8).T.reshape(1, 64)
    W2cm = W2.reshape(8, 8, 7).transpose(1, 0, 2).reshape(64, 7)
    asf2 = att_src2.reshape(1, 7)
    adf2 = att_dst2.reshape(1, 7)
    b2r = b2.reshape(1, 7)

    pad = EPAD - E_EDGES
    srcp = jnp.concatenate([edge_index[0], jnp.zeros((pad,), jnp.int32)])
    dstp = jnp.concatenate([edge_index[1], jnp.zeros((pad,), jnp.int32)])

    tsrc, tdst = _tc_a(x.T, W1cm, attsf, attdf)
    acc1 = _sc_edge1(srcp, dstp, tsrc, tdst)
    t2 = _tc_c(tsrc, tdst, acc1, b1r, W2cm, asf2, adf2)
    acc2 = _sc_edge2(srcp, dstp, t2)
    outp = _tc_e(acc2, t2, b2r)
    return outp[:, :7]
